# Initial kernel scaffold; baseline (speedup 1.0000x reference)
import jax
import jax.numpy as jnp
from jax.experimental import pallas as pl


def kernel(params, word_syn_x, txt_syn_edge_index, txt_sem_x, txt_sem_edge_index, objects, obj_edge_index, obj_pos_x, txt_batch, obj_batch):
    raise NotImplementedError("write your pallas kernel here")


# speedup vs baseline: 23.2626x; 23.2626x over previous
import functools
import math

import jax
import jax.numpy as jnp
from jax import lax
from jax.experimental import pallas as pl
from jax.experimental.pallas import tpu as pltpu
from jax.experimental.pallas import tpu_sc as plsc

TXT_N = 2048
OBJ_N = 10000
NB = 16
NANS = 3129
NANS_PAD = 3200
NC = 2
NS = 16
NW = NC * NS

F32 = jnp.float32


def _dense_pre(x, W, a_s, a_d, br):
    n, di = x.shape
    do = W.shape[1]
    nb = n // br
    a2 = jnp.concatenate(
        [a_s[None, :], a_d[None, :], jnp.zeros((6, do), F32)], axis=0)

    def body(x_ref, w_ref, a_ref, h_ref, aux_ref):
        h = jnp.dot(x_ref[...], w_ref[...], preferred_element_type=F32)
        h_ref[...] = h
        al = jnp.sum(h * a_ref[0, :][None, :], axis=1)
        ar = jnp.sum(h * a_ref[1, :][None, :], axis=1)
        aux_ref[...] = jnp.concatenate(
            [al[:, None], ar[:, None], jnp.zeros((br, 6), F32)], axis=1)

    h, aux = pl.pallas_call(
        body,
        grid=(nb,),
        in_specs=[
            pl.BlockSpec((br, di), lambda i: (i, 0)),
            pl.BlockSpec((di, do), lambda i: (0, 0)),
            pl.BlockSpec((8, do), lambda i: (0, 0)),
        ],
        out_specs=[
            pl.BlockSpec((br, do), lambda i: (i, 0)),
            pl.BlockSpec((br, 8), lambda i: (i, 0)),
        ],
        out_shape=[
            jax.ShapeDtypeStruct((n, do), F32),
            jax.ShapeDtypeStruct((n, 8), F32),
        ],
    )(x, W, a2)
    return h, aux


def _gat_edges(h, al, ar, src, dst, *, n, D, CE, ZR):
    E = src.shape[0]
    EP = E // NW
    NIT = EP // CE
    RP = n // NS
    NZ = RP // ZR

    mesh = plsc.VectorSubcoreMesh(core_axis_name="c", subcore_axis_name="s",
                                  num_cores=NC, num_subcores=NS)

    @functools.partial(
        pl.kernel,
        out_type=[
            jax.ShapeDtypeStruct((NC, n, D), F32),
            jax.ShapeDtypeStruct((NW, n), F32),
        ],
        mesh=mesh,
        compiler_params=pltpu.CompilerParams(
            needs_layout_passes=False, use_tc_tiling_on_sc=False),
        scratch_types=[
            pltpu.VMEM((n,), F32),
            pltpu.VMEM((n,), F32),
            pltpu.VMEM((n,), F32),
            pltpu.VMEM((CE,), jnp.int32),
            pltpu.VMEM((CE,), jnp.int32),
            pltpu.VMEM((CE,), F32),
            pltpu.VMEM((CE, D), F32),
            pltpu.VMEM((ZR, D), F32),
            pltpu.VMEM_SHARED((n, D), F32),
        ],
    )
    def k(h_hbm, al_hbm, ar_hbm, src_hbm, dst_hbm, num_hbm, den_hbm,
          alv, arv, denv, srcv, dstv, exv, rows, zbuf, acc):
        c = lax.axis_index("c")
        s = lax.axis_index("s")
        w = c * NS + s
        zero16 = jnp.zeros((16,), F32)

        pltpu.sync_copy(al_hbm, alv)
        pltpu.sync_copy(ar_hbm, arv)

        def zden(i, carry):
            denv[pl.ds(i * 16, 16)] = zero16
            return carry
        lax.fori_loop(0, n // 16, zden, 0)

        def zzb(r, carry):
            for fg in range(D // 16):
                zbuf[r, pl.ds(fg * 16, 16)] = zero16
            return carry
        lax.fori_loop(0, ZR, zzb, 0)

        def zacc(i, carry):
            pltpu.sync_copy(zbuf, acc.at[pl.ds(s * RP + i * ZR, ZR)])
            return carry
        lax.fori_loop(0, NZ, zacc, 0)

        neg = jnp.full((16,), -3e38, F32)

        def mx(i, carry):
            ca, cr = carry
            ca = jnp.maximum(ca, alv[pl.ds(i * 16, 16)])
            cr = jnp.maximum(cr, arv[pl.ds(i * 16, 16)])
            return ca, cr
        ca, cr = lax.fori_loop(0, n // 16, mx, (neg, neg))
        lanes = lax.iota(jnp.int32, 16)
        for k in (8, 4, 2, 1):
            exv[pl.ds(0, 16)] = ca
            ca = jnp.maximum(ca, plsc.load_gather(exv, [lanes ^ k]))
            exv[pl.ds(0, 16)] = cr
            cr = jnp.maximum(cr, plsc.load_gather(exv, [lanes ^ k]))
        mv = jnp.maximum(ca + cr, jnp.zeros((16,), F32))

        plsc.subcore_barrier()

        def eloop(it, carry):
            base = w * EP + it * CE
            pltpu.sync_copy(src_hbm.at[pl.ds(base, CE)], srcv)
            pltpu.sync_copy(dst_hbm.at[pl.ds(base, CE)], dstv)
            for g in range(CE // 16):
                si = srcv[pl.ds(g * 16, 16)]
                di = dstv[pl.ds(g * 16, 16)]
                t = plsc.load_gather(alv, [si]) + plsc.load_gather(arv, [di])
                e = jnp.where(t > 0.0, t, 0.2 * t)
                ex = jnp.exp(e - mv)
                exv[pl.ds(g * 16, 16)] = ex
                plsc.addupdate_scatter(denv, [di], ex)
            pltpu.sync_copy(h_hbm.at[srcv], rows)

            def scale(e2, carry):
                exb = plsc.load_gather(exv, [jnp.full((16,), e2, jnp.int32)])
                for fg in range(D // 16):
                    rows[e2, pl.ds(fg * 16, 16)] = (
                        rows[e2, pl.ds(fg * 16, 16)] * exb)
                return carry
            lax.fori_loop(0, CE, scale, 0)
            pltpu.sync_copy(rows, acc.at[dstv], add=True)
            return carry
        lax.fori_loop(0, NIT, eloop, 0)

        plsc.subcore_barrier()

        pltpu.sync_copy(denv, den_hbm.at[w])
        pltpu.sync_copy(acc.at[pl.ds(s * RP, RP)],
                        num_hbm.at[c, pl.ds(s * RP, RP)])

    return k(h, al, ar, src, dst)


def _gat_post(num, den, b, br):
    _, n, D = num.shape
    b2 = b[None, :]
    den_t = den.T

    def body(num_ref, den_ref, b_ref, out_ref):
        tot = num_ref[0] + num_ref[1]
        dd = jnp.sum(den_ref[...], axis=1)
        out_ref[...] = (tot / jnp.maximum(dd, 1e-30)[:, None]
                        + b_ref[0, :][None, :])

    return pl.pallas_call(
        body,
        grid=(n // br,),
        in_specs=[
            pl.BlockSpec((NC, br, D), lambda i: (0, i, 0)),
            pl.BlockSpec((br, NW), lambda i: (i, 0)),
            pl.BlockSpec((1, D), lambda i: (0, 0)),
        ],
        out_specs=pl.BlockSpec((br, D), lambda i: (i, 0)),
        out_shape=jax.ShapeDtypeStruct((n, D), F32),
    )(num, den_t, b2)


def _matmul(a, b):
    n, k = a.shape
    d = b.shape[1]

    def body(a_ref, b_ref, o_ref):
        o_ref[...] = jnp.dot(a_ref[...], b_ref[...],
                             preferred_element_type=F32)

    return pl.pallas_call(
        body,
        out_shape=jax.ShapeDtypeStruct((n, d), F32),
    )(a, b)


def _biatt_a(F, Wk, Q, Sm, br):
    n, d = F.shape
    nt = Sm.shape[0]
    nb = n // br
    scale = 1.0 / math.sqrt(float(d))

    def body(f_ref, wk_ref, q_ref, sm_ref, out_ref, cm_ref):
        i = pl.program_id(0)
        K = jnp.dot(f_ref[...], wk_ref[...], preferred_element_type=F32)
        S = lax.dot_general(K, q_ref[...], (((1,), (1,)), ((), ())),
                            preferred_element_type=F32) * scale
        rm = jnp.max(S, axis=1)
        P = jnp.exp(S - rm[:, None])
        rs = jnp.sum(P, axis=1)
        out_ref[...] = f_ref[...] + (
            jnp.dot(P, sm_ref[...], preferred_element_type=F32)
            / rs[:, None])
        bm = jnp.max(S, axis=0)[None, :]

        @pl.when(i == 0)
        def _():
            cm_ref[...] = bm

        @pl.when(i > 0)
        def _():
            cm_ref[...] = jnp.maximum(cm_ref[...], bm)

    return pl.pallas_call(
        body,
        grid=(nb,),
        in_specs=[
            pl.BlockSpec((br, d), lambda i: (i, 0)),
            pl.BlockSpec((d, d), lambda i: (0, 0)),
            pl.BlockSpec((nt, d), lambda i: (0, 0)),
            pl.BlockSpec((nt, d), lambda i: (0, 0)),
        ],
        out_specs=[
            pl.BlockSpec((br, d), lambda i: (i, 0)),
            pl.BlockSpec((1, nt), lambda i: (0, 0)),
        ],
        out_shape=[
            jax.ShapeDtypeStruct((n, d), F32),
            jax.ShapeDtypeStruct((1, nt), F32),
        ],
    )(F, Wk, Q, Sm)


def _biatt_b(F, Wk, Q, Sm, colmax, br):
    n, d = F.shape
    nt = Sm.shape[0]
    nb = n // br
    scale = 1.0 / math.sqrt(float(d))

    def body(f_ref, wk_ref, q_ref, sm_ref, cm_ref, out_ref, nacc, cs):
        i = pl.program_id(0)

        @pl.when(i == 0)
        def _():
            nacc[...] = jnp.zeros_like(nacc)
            cs[...] = jnp.zeros_like(cs)

        K = jnp.dot(f_ref[...], wk_ref[...], preferred_element_type=F32)
        S = lax.dot_general(K, q_ref[...], (((1,), (1,)), ((), ())),
                            preferred_element_type=F32) * scale
        Eexp = jnp.exp(S - cm_ref[...])
        cs[...] += jnp.sum(Eexp, axis=0)[None, :]
        nacc[...] += lax.dot_general(Eexp, f_ref[...],
                                     (((0,), (0,)), ((), ())),
                                     preferred_element_type=F32)

        @pl.when(i == nb - 1)
        def _():
            out_ref[...] = sm_ref[...] + nacc[...] / cs[0, :][:, None]

    return pl.pallas_call(
        body,
        grid=(nb,),
        in_specs=[
            pl.BlockSpec((br, d), lambda i: (i, 0)),
            pl.BlockSpec((d, d), lambda i: (0, 0)),
            pl.BlockSpec((nt, d), lambda i: (0, 0)),
            pl.BlockSpec((nt, d), lambda i: (0, 0)),
            pl.BlockSpec((1, nt), lambda i: (0, 0)),
        ],
        out_specs=pl.BlockSpec((nt, d), lambda i: (0, 0)),
        out_shape=jax.ShapeDtypeStruct((nt, d), F32),
        scratch_shapes=[
            pltpu.VMEM((nt, d), F32),
            pltpu.VMEM((1, nt), F32),
        ],
    )(F, Wk, Q, Sm, colmax)


def _pool(x, batch8, br):
    n, D = x.shape
    nb = n // br

    def body(x_ref, b_ref, out_ref, sums, cnts):
        i = pl.program_id(0)

        @pl.when(i == 0)
        def _():
            sums[...] = jnp.zeros_like(sums)
            cnts[...] = jnp.zeros_like(cnts)

        ids = b_ref[:, 0]
        oh = (lax.broadcasted_iota(jnp.int32, (NB, br), 0)
              == ids[None, :]).astype(F32)
        sums[...] += jnp.dot(oh, x_ref[...], preferred_element_type=F32)
        cnts[...] += jnp.broadcast_to(jnp.sum(oh, axis=1)[:, None], (NB, D))

        @pl.when(i == nb - 1)
        def _():
            out_ref[...] = sums[...] / jnp.maximum(cnts[...], 1.0)

    return pl.pallas_call(
        body,
        grid=(nb,),
        in_specs=[
            pl.BlockSpec((br, D), lambda i: (i, 0)),
            pl.BlockSpec((br, 8), lambda i: (i, 0)),
        ],
        out_specs=pl.BlockSpec((NB, D), lambda i: (0, 0)),
        out_shape=jax.ShapeDtypeStruct((NB, D), F32),
        scratch_shapes=[
            pltpu.VMEM((NB, D), F32),
            pltpu.VMEM((NB, D), F32),
        ],
    )(x, batch8)


def _fusion(fused, W1, b1, W2p, b2p):
    in_dim, hid = W1.shape
    KB = 256
    nkb = hid // KB

    def body(f_ref, w1_ref, b1_ref, w2_ref, b2_ref, out_ref, acc):
        i = pl.program_id(0)

        @pl.when(i == 0)
        def _():
            acc[...] = jnp.broadcast_to(b2_ref[0, :][None, :], acc.shape)

        h1 = (jnp.dot(f_ref[...], w1_ref[...], preferred_element_type=F32)
              + b1_ref[0, :][None, :])
        acc[...] += jnp.dot(h1, w2_ref[...], preferred_element_type=F32)

        @pl.when(i == nkb - 1)
        def _():
            logits = acc[...]
            z = logits - jnp.max(logits, axis=1, keepdims=True)
            out_ref[...] = z - jnp.log(
                jnp.sum(jnp.exp(z), axis=1, keepdims=True))

    return pl.pallas_call(
        body,
        grid=(nkb,),
        in_specs=[
            pl.BlockSpec((NB, in_dim), lambda i: (0, 0)),
            pl.BlockSpec((in_dim, KB), lambda i: (0, i)),
            pl.BlockSpec((1, KB), lambda i: (0, i)),
            pl.BlockSpec((KB, NANS_PAD), lambda i: (i, 0)),
            pl.BlockSpec((1, NANS_PAD), lambda i: (0, 0)),
        ],
        out_specs=pl.BlockSpec((NB, NANS_PAD), lambda i: (0, 0)),
        out_shape=jax.ShapeDtypeStruct((NB, NANS_PAD), F32),
        scratch_shapes=[pltpu.VMEM((NB, NANS_PAD), F32)],
    )(fused, W1, b1, W2p, b2p)


def _gat_layer(x, src, dst, p, *, n, br, CE, ZR, pad_to=None):
    W, a_s, a_d, b = p
    if pad_to is not None and W.shape[1] < pad_to:
        extra = pad_to - W.shape[1]
        W = jnp.pad(W, ((0, 0), (0, extra)))
        a_s = jnp.pad(a_s, (0, extra))
        a_d = jnp.pad(a_d, (0, extra))
        b = jnp.pad(b, (0, extra))
    D = W.shape[1]
    h, aux = _dense_pre(x, W, a_s, a_d, br)
    num, den = _gat_edges(h, aux[:, 0], aux[:, 1], src, dst,
                          n=n, D=D, CE=CE, ZR=ZR)
    return _gat_post(num, den, b, br)


def kernel(params, word_syn_x, txt_syn_edge_index, txt_sem_x,
           txt_sem_edge_index, objects, obj_edge_index, obj_pos_x,
           txt_batch, obj_batch):
    syn, sem, objf, objp = word_syn_x, txt_sem_x, objects, obj_pos_x
    ts_src, ts_dst = txt_syn_edge_index[0], txt_syn_edge_index[1]
    tm_src, tm_dst = txt_sem_edge_index[0], txt_sem_edge_index[1]
    ob_src, ob_dst = obj_edge_index[0], obj_edge_index[1]

    txt_kw = dict(n=TXT_N, br=512, CE=64, ZR=64)
    obj_kw = dict(n=OBJ_N, br=1000, CE=80, ZR=25)

    for lvl in range(3):
        syn = _gat_layer(syn, ts_src, ts_dst, params['syn'][lvl], **txt_kw)
        sem = _gat_layer(sem, tm_src, tm_dst, params['sem'][lvl], **txt_kw)
        objf = _gat_layer(objf, ob_src, ob_dst, params['objf'][lvl],
                          **obj_kw)
        objp = _gat_layer(objp, ob_src, ob_dst, params['objp'][lvl],
                          pad_to=16, **obj_kw)
        Wk, Wq = params['biatt'][lvl]
        Q = _matmul(sem, Wq)
        objf_new, colmax = _biatt_a(objf, Wk, Q, sem, br=1000)
        sem = _biatt_b(objf, Wk, Q, sem, colmax, br=1000)
        objf = objf_new

    txt_b8 = jnp.broadcast_to(txt_batch[:, None], (TXT_N, 8))
    obj_b8 = jnp.broadcast_to(obj_batch[:, None], (OBJ_N, 8))
    syn_p = _pool(syn, txt_b8, br=512)
    sem_p = _pool(sem, txt_b8, br=512)
    objf_p = _pool(objf, obj_b8, br=1000)
    objp_p = _pool(objp, obj_b8, br=1000)[:, :8]

    fused = jnp.concatenate([syn_p, objf_p, sem_p, objp_p], axis=1)
    W1, b1, W2, b2 = params['fusion']
    hid = W1.shape[1]
    hid_pad = ((hid + 255) // 256) * 256
    W1p = jnp.pad(W1, ((0, 0), (0, hid_pad - hid)))
    b1p = jnp.pad(b1, (0, hid_pad - hid))
    W2p = jnp.pad(W2, ((0, hid_pad - hid), (0, NANS_PAD - NANS)))
    b2p = jnp.pad(b2, (0, NANS_PAD - NANS), constant_values=-1e30)
    out = _fusion(fused, W1p, b1p[None, :], W2p, b2p[None, :])
    return out[:, :NANS]


# speedup vs baseline: 33.1601x; 1.4255x over previous
import functools
import math

import jax
import jax.numpy as jnp
from jax import lax
from jax.experimental import pallas as pl
from jax.experimental.pallas import tpu as pltpu
from jax.experimental.pallas import tpu_sc as plsc

TXT_N = 2048
OBJ_N = 10000
NB = 16
NANS = 3129
NANS_PAD = 3200
NC = 2
NS = 16
NW = NC * NS

F32 = jnp.float32


def _dense_pre(x, W, a_s, a_d, br):
    n, di = x.shape
    do = W.shape[1]
    nb = n // br
    a2 = jnp.concatenate(
        [a_s[None, :], a_d[None, :], jnp.zeros((6, do), F32)], axis=0)

    def body(x_ref, w_ref, a_ref, h_ref, aux_ref):
        h = jnp.dot(x_ref[...], w_ref[...], preferred_element_type=F32)
        h_ref[...] = h
        al = jnp.sum(h * a_ref[0, :][None, :], axis=1)
        ar = jnp.sum(h * a_ref[1, :][None, :], axis=1)
        aux_ref[...] = jnp.concatenate(
            [al[:, None], ar[:, None], jnp.zeros((br, 6), F32)], axis=1)

    h, aux = pl.pallas_call(
        body,
        grid=(nb,),
        in_specs=[
            pl.BlockSpec((br, di), lambda i: (i, 0)),
            pl.BlockSpec((di, do), lambda i: (0, 0)),
            pl.BlockSpec((8, do), lambda i: (0, 0)),
        ],
        out_specs=[
            pl.BlockSpec((br, do), lambda i: (i, 0)),
            pl.BlockSpec((br, 8), lambda i: (i, 0)),
        ],
        out_shape=[
            jax.ShapeDtypeStruct((n, do), F32),
            jax.ShapeDtypeStruct((n, 8), F32),
        ],
    )(x, W, a2)
    return h, aux


def _gat_edges(h, al, ar, src, dst, *, n, D, CE, ZR):
    E = src.shape[0]
    EP = E // NW
    NIT = EP // CE
    RP = n // NS
    NZ = RP // ZR

    mesh = plsc.VectorSubcoreMesh(core_axis_name="c", subcore_axis_name="s",
                                  num_cores=NC, num_subcores=NS)

    @functools.partial(
        pl.kernel,
        out_type=[
            jax.ShapeDtypeStruct((NC, n, D), F32),
            jax.ShapeDtypeStruct((NW, n), F32),
        ],
        mesh=mesh,
        compiler_params=pltpu.CompilerParams(
            needs_layout_passes=False, use_tc_tiling_on_sc=False),
        scratch_types=[
            pltpu.VMEM((n,), F32),
            pltpu.VMEM((n,), F32),
            pltpu.VMEM((n,), F32),
            pltpu.VMEM((2, CE), jnp.int32),
            pltpu.VMEM((2, CE), jnp.int32),
            pltpu.VMEM((CE,), F32),
            pltpu.VMEM((CE, D), F32),
            pltpu.VMEM((ZR, D), F32),
            pltpu.VMEM_SHARED((n, D), F32),
            pltpu.SemaphoreType.DMA,
            pltpu.SemaphoreType.DMA,
        ],
    )
    def k(h_hbm, al_hbm, ar_hbm, src_hbm, dst_hbm, num_hbm, den_hbm,
          alv, arv, denv, srcv, dstv, exv, rows, zbuf, acc, sem_i, sem_g):
        c = lax.axis_index("c")
        s = lax.axis_index("s")
        w = c * NS + s
        zero16 = jnp.zeros((16,), F32)

        pltpu.sync_copy(al_hbm, alv)
        pltpu.sync_copy(ar_hbm, arv)

        def zden(i, carry):
            denv[pl.ds(i * 16, 16)] = zero16
            return carry
        lax.fori_loop(0, n // 16, zden, 0)

        def zzb(r, carry):
            for fg in range(D // 16):
                zbuf[r, pl.ds(fg * 16, 16)] = zero16
            return carry
        lax.fori_loop(0, ZR, zzb, 0)

        def zacc(i, carry):
            pltpu.sync_copy(zbuf, acc.at[pl.ds(s * RP + i * ZR, ZR)])
            return carry
        lax.fori_loop(0, NZ, zacc, 0)

        neg = jnp.full((16,), -3e38, F32)

        def mx(i, carry):
            ca, cr = carry
            ca = jnp.maximum(ca, alv[pl.ds(i * 16, 16)])
            cr = jnp.maximum(cr, arv[pl.ds(i * 16, 16)])
            return ca, cr
        ca, cr = lax.fori_loop(0, n // 16, mx, (neg, neg))
        lanes = lax.iota(jnp.int32, 16)
        for k in (8, 4, 2, 1):
            exv[pl.ds(0, 16)] = ca
            ca = jnp.maximum(ca, plsc.load_gather(exv, [lanes ^ k]))
            exv[pl.ds(0, 16)] = cr
            cr = jnp.maximum(cr, plsc.load_gather(exv, [lanes ^ k]))
        mv = jnp.maximum(ca + cr, jnp.zeros((16,), F32))

        plsc.subcore_barrier()

        def fetch_idx(it, buf):
            base = w * EP + it * CE
            pltpu.async_copy(src_hbm.at[pl.ds(base, CE)], srcv.at[buf], sem_i)
            pltpu.async_copy(dst_hbm.at[pl.ds(base, CE)], dstv.at[buf], sem_i)

        def wait_idx(buf):
            pltpu.make_async_copy(
                src_hbm.at[pl.ds(0, CE)], srcv.at[buf], sem_i).wait()
            pltpu.make_async_copy(
                src_hbm.at[pl.ds(0, CE)], dstv.at[buf], sem_i).wait()

        def process(it, buf):
            gd = pltpu.async_copy(h_hbm.at[srcv.at[buf]], rows, sem_g)
            for g in range(CE // 16):
                si = srcv[buf, pl.ds(g * 16, 16)]
                di = dstv[buf, pl.ds(g * 16, 16)]
                t = plsc.load_gather(alv, [si]) + plsc.load_gather(arv, [di])
                e = jnp.where(t > 0.0, t, 0.2 * t)
                ex = jnp.exp(e - mv)
                exv[pl.ds(g * 16, 16)] = ex
                plsc.addupdate_scatter(denv, [di], ex)
            gd.wait()

            def scale(g, carry):
                for l in range(16):
                    eidx = g * 16 + l
                    exb = plsc.load_gather(
                        exv, [jnp.full((16,), eidx, jnp.int32)])
                    for fg in range(D // 16):
                        rows[eidx, pl.ds(fg * 16, 16)] = (
                            rows[eidx, pl.ds(fg * 16, 16)] * exb)
                return carry
            lax.fori_loop(0, CE // 16, scale, 0)
            pltpu.sync_copy(rows, acc.at[dstv.at[buf]], add=True)

        fetch_idx(0, 0)

        def body2(j, carry):
            it0 = j * 2
            wait_idx(0)
            fetch_idx(it0 + 1, 1)
            process(it0, 0)
            wait_idx(1)

            @pl.when(it0 + 2 < NIT)
            def _():
                fetch_idx(it0 + 2, 0)
            process(it0 + 1, 1)
            return carry
        lax.fori_loop(0, NIT // 2, body2, 0)
        if NIT % 2 == 1:
            wait_idx(0)
            process(NIT - 1, 0)

        plsc.subcore_barrier()

        pltpu.sync_copy(denv, den_hbm.at[w])
        pltpu.sync_copy(acc.at[pl.ds(s * RP, RP)],
                        num_hbm.at[c, pl.ds(s * RP, RP)])

    return k(h, al, ar, src, dst)


def _gat_post(num, den, b, br):
    _, n, D = num.shape
    b2 = b[None, :]
    den_t = den.T

    def body(num_ref, den_ref, b_ref, out_ref):
        tot = num_ref[0] + num_ref[1]
        dd = jnp.sum(den_ref[...], axis=1)
        out_ref[...] = (tot / jnp.maximum(dd, 1e-30)[:, None]
                        + b_ref[0, :][None, :])

    return pl.pallas_call(
        body,
        grid=(n // br,),
        in_specs=[
            pl.BlockSpec((NC, br, D), lambda i: (0, i, 0)),
            pl.BlockSpec((br, NW), lambda i: (i, 0)),
            pl.BlockSpec((1, D), lambda i: (0, 0)),
        ],
        out_specs=pl.BlockSpec((br, D), lambda i: (i, 0)),
        out_shape=jax.ShapeDtypeStruct((n, D), F32),
    )(num, den_t, b2)


def _matmul(a, b):
    n, k = a.shape
    d = b.shape[1]

    def body(a_ref, b_ref, o_ref):
        o_ref[...] = jnp.dot(a_ref[...], b_ref[...],
                             preferred_element_type=F32)

    return pl.pallas_call(
        body,
        out_shape=jax.ShapeDtypeStruct((n, d), F32),
    )(a, b)


def _biatt_a(F, Wk, Q, Sm, br):
    n, d = F.shape
    nt = Sm.shape[0]
    nb = n // br
    scale = 1.0 / math.sqrt(float(d))

    def body(f_ref, wk_ref, q_ref, sm_ref, out_ref, cm_ref):
        i = pl.program_id(0)
        K = jnp.dot(f_ref[...], wk_ref[...], preferred_element_type=F32)
        S = lax.dot_general(K, q_ref[...], (((1,), (1,)), ((), ())),
                            preferred_element_type=F32) * scale
        rm = jnp.max(S, axis=1)
        P = jnp.exp(S - rm[:, None])
        rs = jnp.sum(P, axis=1)
        out_ref[...] = f_ref[...] + (
            jnp.dot(P, sm_ref[...], preferred_element_type=F32)
            / rs[:, None])
        bm = jnp.max(S, axis=0)[None, :]

        @pl.when(i == 0)
        def _():
            cm_ref[...] = bm

        @pl.when(i > 0)
        def _():
            cm_ref[...] = jnp.maximum(cm_ref[...], bm)

    return pl.pallas_call(
        body,
        grid=(nb,),
        in_specs=[
            pl.BlockSpec((br, d), lambda i: (i, 0)),
            pl.BlockSpec((d, d), lambda i: (0, 0)),
            pl.BlockSpec((nt, d), lambda i: (0, 0)),
            pl.BlockSpec((nt, d), lambda i: (0, 0)),
        ],
        out_specs=[
            pl.BlockSpec((br, d), lambda i: (i, 0)),
            pl.BlockSpec((1, nt), lambda i: (0, 0)),
        ],
        out_shape=[
            jax.ShapeDtypeStruct((n, d), F32),
            jax.ShapeDtypeStruct((1, nt), F32),
        ],
    )(F, Wk, Q, Sm)


def _biatt_b(F, Wk, Q, Sm, colmax, br):
    n, d = F.shape
    nt = Sm.shape[0]
    nb = n // br
    scale = 1.0 / math.sqrt(float(d))

    def body(f_ref, wk_ref, q_ref, sm_ref, cm_ref, out_ref, nacc, cs):
        i = pl.program_id(0)

        @pl.when(i == 0)
        def _():
            nacc[...] = jnp.zeros_like(nacc)
            cs[...] = jnp.zeros_like(cs)

        K = jnp.dot(f_ref[...], wk_ref[...], preferred_element_type=F32)
        S = lax.dot_general(K, q_ref[...], (((1,), (1,)), ((), ())),
                            preferred_element_type=F32) * scale
        Eexp = jnp.exp(S - cm_ref[...])
        cs[...] += jnp.sum(Eexp, axis=0)[None, :]
        nacc[...] += lax.dot_general(Eexp, f_ref[...],
                                     (((0,), (0,)), ((), ())),
                                     preferred_element_type=F32)

        @pl.when(i == nb - 1)
        def _():
            out_ref[...] = sm_ref[...] + nacc[...] / cs[0, :][:, None]

    return pl.pallas_call(
        body,
        grid=(nb,),
        in_specs=[
            pl.BlockSpec((br, d), lambda i: (i, 0)),
            pl.BlockSpec((d, d), lambda i: (0, 0)),
            pl.BlockSpec((nt, d), lambda i: (0, 0)),
            pl.BlockSpec((nt, d), lambda i: (0, 0)),
            pl.BlockSpec((1, nt), lambda i: (0, 0)),
        ],
        out_specs=pl.BlockSpec((nt, d), lambda i: (0, 0)),
        out_shape=jax.ShapeDtypeStruct((nt, d), F32),
        scratch_shapes=[
            pltpu.VMEM((nt, d), F32),
            pltpu.VMEM((1, nt), F32),
        ],
    )(F, Wk, Q, Sm, colmax)


def _pool(x, batch8, br):
    n, D = x.shape
    nb = n // br

    def body(x_ref, b_ref, out_ref, sums, cnts):
        i = pl.program_id(0)

        @pl.when(i == 0)
        def _():
            sums[...] = jnp.zeros_like(sums)
            cnts[...] = jnp.zeros_like(cnts)

        ids = b_ref[:, 0]
        oh = (lax.broadcasted_iota(jnp.int32, (NB, br), 0)
              == ids[None, :]).astype(F32)
        sums[...] += jnp.dot(oh, x_ref[...], preferred_element_type=F32)
        cnts[...] += jnp.broadcast_to(jnp.sum(oh, axis=1)[:, None], (NB, D))

        @pl.when(i == nb - 1)
        def _():
            out_ref[...] = sums[...] / jnp.maximum(cnts[...], 1.0)

    return pl.pallas_call(
        body,
        grid=(nb,),
        in_specs=[
            pl.BlockSpec((br, D), lambda i: (i, 0)),
            pl.BlockSpec((br, 8), lambda i: (i, 0)),
        ],
        out_specs=pl.BlockSpec((NB, D), lambda i: (0, 0)),
        out_shape=jax.ShapeDtypeStruct((NB, D), F32),
        scratch_shapes=[
            pltpu.VMEM((NB, D), F32),
            pltpu.VMEM((NB, D), F32),
        ],
    )(x, batch8)


def _fusion(fused, W1, b1, W2p, b2p):
    in_dim, hid = W1.shape
    KB = 256
    nkb = hid // KB

    def body(f_ref, w1_ref, b1_ref, w2_ref, b2_ref, out_ref, acc):
        i = pl.program_id(0)

        @pl.when(i == 0)
        def _():
            acc[...] = jnp.broadcast_to(b2_ref[0, :][None, :], acc.shape)

        h1 = (jnp.dot(f_ref[...], w1_ref[...], preferred_element_type=F32)
              + b1_ref[0, :][None, :])
        acc[...] += jnp.dot(h1, w2_ref[...], preferred_element_type=F32)

        @pl.when(i == nkb - 1)
        def _():
            logits = acc[...]
            z = logits - jnp.max(logits, axis=1, keepdims=True)
            out_ref[...] = z - jnp.log(
                jnp.sum(jnp.exp(z), axis=1, keepdims=True))

    return pl.pallas_call(
        body,
        grid=(nkb,),
        in_specs=[
            pl.BlockSpec((NB, in_dim), lambda i: (0, 0)),
            pl.BlockSpec((in_dim, KB), lambda i: (0, i)),
            pl.BlockSpec((1, KB), lambda i: (0, i)),
            pl.BlockSpec((KB, NANS_PAD), lambda i: (i, 0)),
            pl.BlockSpec((1, NANS_PAD), lambda i: (0, 0)),
        ],
        out_specs=pl.BlockSpec((NB, NANS_PAD), lambda i: (0, 0)),
        out_shape=jax.ShapeDtypeStruct((NB, NANS_PAD), F32),
        scratch_shapes=[pltpu.VMEM((NB, NANS_PAD), F32)],
    )(fused, W1, b1, W2p, b2p)


def _gat_layer(x, src, dst, p, *, n, br, CE, ZR, pad_to=None):
    W, a_s, a_d, b = p
    if pad_to is not None and W.shape[1] < pad_to:
        extra = pad_to - W.shape[1]
        W = jnp.pad(W, ((0, 0), (0, extra)))
        a_s = jnp.pad(a_s, (0, extra))
        a_d = jnp.pad(a_d, (0, extra))
        b = jnp.pad(b, (0, extra))
    D = W.shape[1]
    h, aux = _dense_pre(x, W, a_s, a_d, br)
    num, den = _gat_edges(h, aux[:, 0], aux[:, 1], src, dst,
                          n=n, D=D, CE=CE, ZR=ZR)
    return _gat_post(num, den, b, br)


def kernel(params, word_syn_x, txt_syn_edge_index, txt_sem_x,
           txt_sem_edge_index, objects, obj_edge_index, obj_pos_x,
           txt_batch, obj_batch):
    syn, sem, objf, objp = word_syn_x, txt_sem_x, objects, obj_pos_x
    ts_src, ts_dst = txt_syn_edge_index[0], txt_syn_edge_index[1]
    tm_src, tm_dst = txt_sem_edge_index[0], txt_sem_edge_index[1]
    ob_src, ob_dst = obj_edge_index[0], obj_edge_index[1]

    txt_kw = dict(n=TXT_N, br=512, CE=64, ZR=64)
    obj_kw = dict(n=OBJ_N, br=1000, CE=80, ZR=25)

    for lvl in range(3):
        syn = _gat_layer(syn, ts_src, ts_dst, params['syn'][lvl], **txt_kw)
        sem = _gat_layer(sem, tm_src, tm_dst, params['sem'][lvl], **txt_kw)
        objf = _gat_layer(objf, ob_src, ob_dst, params['objf'][lvl],
                          **obj_kw)
        objp = _gat_layer(objp, ob_src, ob_dst, params['objp'][lvl],
                          pad_to=16, **obj_kw)
        Wk, Wq = params['biatt'][lvl]
        Q = _matmul(sem, Wq)
        objf_new, colmax = _biatt_a(objf, Wk, Q, sem, br=1000)
        sem = _biatt_b(objf, Wk, Q, sem, colmax, br=1000)
        objf = objf_new

    txt_b8 = jnp.broadcast_to(txt_batch[:, None], (TXT_N, 8))
    obj_b8 = jnp.broadcast_to(obj_batch[:, None], (OBJ_N, 8))
    syn_p = _pool(syn, txt_b8, br=512)
    sem_p = _pool(sem, txt_b8, br=512)
    objf_p = _pool(objf, obj_b8, br=1000)
    objp_p = _pool(objp, obj_b8, br=1000)[:, :8]

    fused = jnp.concatenate([syn_p, objf_p, sem_p, objp_p], axis=1)
    W1, b1, W2, b2 = params['fusion']
    hid = W1.shape[1]
    hid_pad = ((hid + 255) // 256) * 256
    W1p = jnp.pad(W1, ((0, 0), (0, hid_pad - hid)))
    b1p = jnp.pad(b1, (0, hid_pad - hid))
    W2p = jnp.pad(W2, ((0, hid_pad - hid), (0, NANS_PAD - NANS)))
    b2p = jnp.pad(b2, (0, NANS_PAD - NANS), constant_values=-1e30)
    out = _fusion(fused, W1p, b1p[None, :], W2p, b2p[None, :])
    return out[:, :NANS]


# speedup vs baseline: 39.0345x; 1.1772x over previous
import functools
import math

import jax
import jax.numpy as jnp
from jax import lax
from jax.experimental import pallas as pl
from jax.experimental.pallas import tpu as pltpu
from jax.experimental.pallas import tpu_sc as plsc

TXT_N = 2048
OBJ_N = 10000
NB = 16
NANS = 3129
NANS_PAD = 3200
NC = 2
NS = 16
NW = NC * NS

F32 = jnp.float32


def _dense_pre(x, W, a_s, a_d, br):
    n, di = x.shape
    do = W.shape[1]
    nb = n // br
    a2 = jnp.concatenate(
        [a_s[None, :], a_d[None, :], jnp.zeros((6, do), F32)], axis=0)

    def body(x_ref, w_ref, a_ref, h_ref, aux_ref):
        h = jnp.dot(x_ref[...], w_ref[...], preferred_element_type=F32)
        h_ref[...] = h
        al = jnp.sum(h * a_ref[0, :][None, :], axis=1)
        ar = jnp.sum(h * a_ref[1, :][None, :], axis=1)
        aux_ref[...] = jnp.concatenate(
            [al[:, None], ar[:, None], jnp.zeros((br, 6), F32)], axis=1)

    h, aux = pl.pallas_call(
        body,
        grid=(nb,),
        in_specs=[
            pl.BlockSpec((br, di), lambda i: (i, 0)),
            pl.BlockSpec((di, do), lambda i: (0, 0)),
            pl.BlockSpec((8, do), lambda i: (0, 0)),
        ],
        out_specs=[
            pl.BlockSpec((br, do), lambda i: (i, 0)),
            pl.BlockSpec((br, 8), lambda i: (i, 0)),
        ],
        out_shape=[
            jax.ShapeDtypeStruct((n, do), F32),
            jax.ShapeDtypeStruct((n, 8), F32),
        ],
    )(x, W, a2)
    return h, aux


def _gat_edges(h, al, ar, src, dst, *, n, D, CE, ZR):
    E = src.shape[0]
    EP = E // NW
    NIT = EP // CE
    RP = n // NS
    NZ = RP // ZR

    mesh = plsc.VectorSubcoreMesh(core_axis_name="c", subcore_axis_name="s",
                                  num_cores=NC, num_subcores=NS)

    @functools.partial(
        pl.kernel,
        out_type=[
            jax.ShapeDtypeStruct((NC, n, D), F32),
            jax.ShapeDtypeStruct((NW, n), F32),
        ],
        mesh=mesh,
        compiler_params=pltpu.CompilerParams(
            needs_layout_passes=False, use_tc_tiling_on_sc=False),
        scratch_types=[
            pltpu.VMEM((n,), F32),
            pltpu.VMEM((n,), F32),
            pltpu.VMEM((n,), F32),
            pltpu.VMEM((2, CE), jnp.int32),
            pltpu.VMEM((2, CE), jnp.int32),
            pltpu.VMEM((CE,), F32),
            pltpu.VMEM((CE, D), F32),
            pltpu.VMEM((ZR, D), F32),
            pltpu.VMEM_SHARED((n, D), F32),
            pltpu.SemaphoreType.DMA,
            pltpu.SemaphoreType.DMA,
        ],
    )
    def k(h_hbm, al_hbm, ar_hbm, src_hbm, dst_hbm, num_hbm, den_hbm,
          alv, arv, denv, srcv, dstv, exv, rows, zbuf, acc, sem_i, sem_g):
        c = lax.axis_index("c")
        s = lax.axis_index("s")
        w = c * NS + s
        zero16 = jnp.zeros((16,), F32)

        pltpu.sync_copy(al_hbm, alv)
        pltpu.sync_copy(ar_hbm, arv)

        def zden(i, carry):
            denv[pl.ds(i * 16, 16)] = zero16
            return carry
        lax.fori_loop(0, n // 16, zden, 0)

        def zzb(r, carry):
            for fg in range(D // 16):
                zbuf[r, pl.ds(fg * 16, 16)] = zero16
            return carry
        lax.fori_loop(0, ZR, zzb, 0)

        def zacc(i, carry):
            pltpu.sync_copy(zbuf, acc.at[pl.ds(s * RP + i * ZR, ZR)])
            return carry
        lax.fori_loop(0, NZ, zacc, 0)

        neg = jnp.full((16,), -3e38, F32)

        def mx(i, carry):
            ca, cr = carry
            ca = jnp.maximum(ca, alv[pl.ds(i * 16, 16)])
            cr = jnp.maximum(cr, arv[pl.ds(i * 16, 16)])
            return ca, cr
        ca, cr = lax.fori_loop(0, n // 16, mx, (neg, neg))
        lanes = lax.iota(jnp.int32, 16)
        for k in (8, 4, 2, 1):
            exv[pl.ds(0, 16)] = ca
            ca = jnp.maximum(ca, plsc.load_gather(exv, [lanes ^ k]))
            exv[pl.ds(0, 16)] = cr
            cr = jnp.maximum(cr, plsc.load_gather(exv, [lanes ^ k]))
        mv = jnp.maximum(ca + cr, jnp.zeros((16,), F32))

        plsc.subcore_barrier()

        def fetch_idx(it, buf):
            base = w * EP + it * CE
            pltpu.async_copy(src_hbm.at[pl.ds(base, CE)], srcv.at[buf], sem_i)
            pltpu.async_copy(dst_hbm.at[pl.ds(base, CE)], dstv.at[buf], sem_i)

        def wait_idx(buf):
            pltpu.make_async_copy(
                src_hbm.at[pl.ds(0, CE)], srcv.at[buf], sem_i).wait()
            pltpu.make_async_copy(
                src_hbm.at[pl.ds(0, CE)], dstv.at[buf], sem_i).wait()

        def process(it, buf):
            gd = pltpu.async_copy(h_hbm.at[srcv.at[buf]], rows, sem_g)
            for g in range(CE // 16):
                si = srcv[buf, pl.ds(g * 16, 16)]
                di = dstv[buf, pl.ds(g * 16, 16)]
                t = plsc.load_gather(alv, [si]) + plsc.load_gather(arv, [di])
                e = jnp.where(t > 0.0, t, 0.2 * t)
                ex = jnp.exp(e - mv)
                exv[pl.ds(g * 16, 16)] = ex
                plsc.addupdate_scatter(denv, [di], ex)
            gd.wait()

            def scale(g, carry):
                exg = exv[pl.ds(g * 16, 16)]
                for l in range(16):
                    eidx = g * 16 + l
                    exb = jnp.full((16,), exg[l], F32)
                    for fg in range(D // 16):
                        rows[eidx, pl.ds(fg * 16, 16)] = (
                            rows[eidx, pl.ds(fg * 16, 16)] * exb)
                return carry
            lax.fori_loop(0, CE // 16, scale, 0)
            pltpu.sync_copy(rows, acc.at[dstv.at[buf]], add=True)

        fetch_idx(0, 0)

        def body2(j, carry):
            it0 = j * 2
            wait_idx(0)
            fetch_idx(it0 + 1, 1)
            process(it0, 0)
            wait_idx(1)

            @pl.when(it0 + 2 < NIT)
            def _():
                fetch_idx(it0 + 2, 0)
            process(it0 + 1, 1)
            return carry
        lax.fori_loop(0, NIT // 2, body2, 0)
        if NIT % 2 == 1:
            wait_idx(0)
            process(NIT - 1, 0)

        plsc.subcore_barrier()

        pltpu.sync_copy(denv, den_hbm.at[w])
        pltpu.sync_copy(acc.at[pl.ds(s * RP, RP)],
                        num_hbm.at[c, pl.ds(s * RP, RP)])

    return k(h, al, ar, src, dst)


def _gat_post(num, den, b, br):
    _, n, D = num.shape
    b2 = b[None, :]
    den_t = den.T

    def body(num_ref, den_ref, b_ref, out_ref):
        tot = num_ref[0] + num_ref[1]
        dd = jnp.sum(den_ref[...], axis=1)
        out_ref[...] = (tot / jnp.maximum(dd, 1e-30)[:, None]
                        + b_ref[0, :][None, :])

    return pl.pallas_call(
        body,
        grid=(n // br,),
        in_specs=[
            pl.BlockSpec((NC, br, D), lambda i: (0, i, 0)),
            pl.BlockSpec((br, NW), lambda i: (i, 0)),
            pl.BlockSpec((1, D), lambda i: (0, 0)),
        ],
        out_specs=pl.BlockSpec((br, D), lambda i: (i, 0)),
        out_shape=jax.ShapeDtypeStruct((n, D), F32),
    )(num, den_t, b2)


def _matmul(a, b):
    n, k = a.shape
    d = b.shape[1]

    def body(a_ref, b_ref, o_ref):
        o_ref[...] = jnp.dot(a_ref[...], b_ref[...],
                             preferred_element_type=F32)

    return pl.pallas_call(
        body,
        out_shape=jax.ShapeDtypeStruct((n, d), F32),
    )(a, b)


def _biatt_a(F, Wk, Q, Sm, br):
    n, d = F.shape
    nt = Sm.shape[0]
    nb = n // br
    scale = 1.0 / math.sqrt(float(d))

    def body(f_ref, wk_ref, q_ref, sm_ref, out_ref, cm_ref):
        i = pl.program_id(0)
        K = jnp.dot(f_ref[...], wk_ref[...], preferred_element_type=F32)
        S = lax.dot_general(K, q_ref[...], (((1,), (1,)), ((), ())),
                            preferred_element_type=F32) * scale
        rm = jnp.max(S, axis=1)
        P = jnp.exp(S - rm[:, None])
        rs = jnp.sum(P, axis=1)
        out_ref[...] = f_ref[...] + (
            jnp.dot(P, sm_ref[...], preferred_element_type=F32)
            / rs[:, None])
        bm = jnp.max(S, axis=0)[None, :]

        @pl.when(i == 0)
        def _():
            cm_ref[...] = bm

        @pl.when(i > 0)
        def _():
            cm_ref[...] = jnp.maximum(cm_ref[...], bm)

    return pl.pallas_call(
        body,
        grid=(nb,),
        in_specs=[
            pl.BlockSpec((br, d), lambda i: (i, 0)),
            pl.BlockSpec((d, d), lambda i: (0, 0)),
            pl.BlockSpec((nt, d), lambda i: (0, 0)),
            pl.BlockSpec((nt, d), lambda i: (0, 0)),
        ],
        out_specs=[
            pl.BlockSpec((br, d), lambda i: (i, 0)),
            pl.BlockSpec((1, nt), lambda i: (0, 0)),
        ],
        out_shape=[
            jax.ShapeDtypeStruct((n, d), F32),
            jax.ShapeDtypeStruct((1, nt), F32),
        ],
    )(F, Wk, Q, Sm)


def _biatt_b(F, Wk, Q, Sm, colmax, br):
    n, d = F.shape
    nt = Sm.shape[0]
    nb = n // br
    scale = 1.0 / math.sqrt(float(d))

    def body(f_ref, wk_ref, q_ref, sm_ref, cm_ref, out_ref, nacc, cs):
        i = pl.program_id(0)

        @pl.when(i == 0)
        def _():
            nacc[...] = jnp.zeros_like(nacc)
            cs[...] = jnp.zeros_like(cs)

        K = jnp.dot(f_ref[...], wk_ref[...], preferred_element_type=F32)
        S = lax.dot_general(K, q_ref[...], (((1,), (1,)), ((), ())),
                            preferred_element_type=F32) * scale
        Eexp = jnp.exp(S - cm_ref[...])
        cs[...] += jnp.sum(Eexp, axis=0)[None, :]
        nacc[...] += lax.dot_general(Eexp, f_ref[...],
                                     (((0,), (0,)), ((), ())),
                                     preferred_element_type=F32)

        @pl.when(i == nb - 1)
        def _():
            out_ref[...] = sm_ref[...] + nacc[...] / cs[0, :][:, None]

    return pl.pallas_call(
        body,
        grid=(nb,),
        in_specs=[
            pl.BlockSpec((br, d), lambda i: (i, 0)),
            pl.BlockSpec((d, d), lambda i: (0, 0)),
            pl.BlockSpec((nt, d), lambda i: (0, 0)),
            pl.BlockSpec((nt, d), lambda i: (0, 0)),
            pl.BlockSpec((1, nt), lambda i: (0, 0)),
        ],
        out_specs=pl.BlockSpec((nt, d), lambda i: (0, 0)),
        out_shape=jax.ShapeDtypeStruct((nt, d), F32),
        scratch_shapes=[
            pltpu.VMEM((nt, d), F32),
            pltpu.VMEM((1, nt), F32),
        ],
    )(F, Wk, Q, Sm, colmax)


def _pool(x, batch8, br):
    n, D = x.shape
    nb = n // br

    def body(x_ref, b_ref, out_ref, sums, cnts):
        i = pl.program_id(0)

        @pl.when(i == 0)
        def _():
            sums[...] = jnp.zeros_like(sums)
            cnts[...] = jnp.zeros_like(cnts)

        ids = b_ref[:, 0]
        oh = (lax.broadcasted_iota(jnp.int32, (NB, br), 0)
              == ids[None, :]).astype(F32)
        sums[...] += jnp.dot(oh, x_ref[...], preferred_element_type=F32)
        cnts[...] += jnp.broadcast_to(jnp.sum(oh, axis=1)[:, None], (NB, D))

        @pl.when(i == nb - 1)
        def _():
            out_ref[...] = sums[...] / jnp.maximum(cnts[...], 1.0)

    return pl.pallas_call(
        body,
        grid=(nb,),
        in_specs=[
            pl.BlockSpec((br, D), lambda i: (i, 0)),
            pl.BlockSpec((br, 8), lambda i: (i, 0)),
        ],
        out_specs=pl.BlockSpec((NB, D), lambda i: (0, 0)),
        out_shape=jax.ShapeDtypeStruct((NB, D), F32),
        scratch_shapes=[
            pltpu.VMEM((NB, D), F32),
            pltpu.VMEM((NB, D), F32),
        ],
    )(x, batch8)


def _fusion(fused, W1, b1, W2p, b2p):
    in_dim, hid = W1.shape
    KB = 256
    nkb = hid // KB

    def body(f_ref, w1_ref, b1_ref, w2_ref, b2_ref, out_ref, acc):
        i = pl.program_id(0)

        @pl.when(i == 0)
        def _():
            acc[...] = jnp.broadcast_to(b2_ref[0, :][None, :], acc.shape)

        h1 = (jnp.dot(f_ref[...], w1_ref[...], preferred_element_type=F32)
              + b1_ref[0, :][None, :])
        acc[...] += jnp.dot(h1, w2_ref[...], preferred_element_type=F32)

        @pl.when(i == nkb - 1)
        def _():
            logits = acc[...]
            z = logits - jnp.max(logits, axis=1, keepdims=True)
            out_ref[...] = z - jnp.log(
                jnp.sum(jnp.exp(z), axis=1, keepdims=True))

    return pl.pallas_call(
        body,
        grid=(nkb,),
        in_specs=[
            pl.BlockSpec((NB, in_dim), lambda i: (0, 0)),
            pl.BlockSpec((in_dim, KB), lambda i: (0, i)),
            pl.BlockSpec((1, KB), lambda i: (0, i)),
            pl.BlockSpec((KB, NANS_PAD), lambda i: (i, 0)),
            pl.BlockSpec((1, NANS_PAD), lambda i: (0, 0)),
        ],
        out_specs=pl.BlockSpec((NB, NANS_PAD), lambda i: (0, 0)),
        out_shape=jax.ShapeDtypeStruct((NB, NANS_PAD), F32),
        scratch_shapes=[pltpu.VMEM((NB, NANS_PAD), F32)],
    )(fused, W1, b1, W2p, b2p)


def _gat_layer(x, src, dst, p, *, n, br, CE, ZR, pad_to=None):
    W, a_s, a_d, b = p
    if pad_to is not None and W.shape[1] < pad_to:
        extra = pad_to - W.shape[1]
        W = jnp.pad(W, ((0, 0), (0, extra)))
        a_s = jnp.pad(a_s, (0, extra))
        a_d = jnp.pad(a_d, (0, extra))
        b = jnp.pad(b, (0, extra))
    D = W.shape[1]
    h, aux = _dense_pre(x, W, a_s, a_d, br)
    num, den = _gat_edges(h, aux[:, 0], aux[:, 1], src, dst,
                          n=n, D=D, CE=CE, ZR=ZR)
    return _gat_post(num, den, b, br)


def kernel(params, word_syn_x, txt_syn_edge_index, txt_sem_x,
           txt_sem_edge_index, objects, obj_edge_index, obj_pos_x,
           txt_batch, obj_batch):
    syn, sem, objf, objp = word_syn_x, txt_sem_x, objects, obj_pos_x
    ts_src, ts_dst = txt_syn_edge_index[0], txt_syn_edge_index[1]
    tm_src, tm_dst = txt_sem_edge_index[0], txt_sem_edge_index[1]
    ob_src, ob_dst = obj_edge_index[0], obj_edge_index[1]

    txt_kw = dict(n=TXT_N, br=512, CE=64, ZR=64)
    obj_kw = dict(n=OBJ_N, br=1000, CE=80, ZR=25)

    for lvl in range(3):
        syn = _gat_layer(syn, ts_src, ts_dst, params['syn'][lvl], **txt_kw)
        sem = _gat_layer(sem, tm_src, tm_dst, params['sem'][lvl], **txt_kw)
        objf = _gat_layer(objf, ob_src, ob_dst, params['objf'][lvl],
                          **obj_kw)
        objp = _gat_layer(objp, ob_src, ob_dst, params['objp'][lvl],
                          pad_to=16, **obj_kw)
        Wk, Wq = params['biatt'][lvl]
        Q = _matmul(sem, Wq)
        objf_new, colmax = _biatt_a(objf, Wk, Q, sem, br=1000)
        sem = _biatt_b(objf, Wk, Q, sem, colmax, br=1000)
        objf = objf_new

    txt_b8 = jnp.broadcast_to(txt_batch[:, None], (TXT_N, 8))
    obj_b8 = jnp.broadcast_to(obj_batch[:, None], (OBJ_N, 8))
    syn_p = _pool(syn, txt_b8, br=512)
    sem_p = _pool(sem, txt_b8, br=512)
    objf_p = _pool(objf, obj_b8, br=1000)
    objp_p = _pool(objp, obj_b8, br=1000)[:, :8]

    fused = jnp.concatenate([syn_p, objf_p, sem_p, objp_p], axis=1)
    W1, b1, W2, b2 = params['fusion']
    hid = W1.shape[1]
    hid_pad = ((hid + 255) // 256) * 256
    W1p = jnp.pad(W1, ((0, 0), (0, hid_pad - hid)))
    b1p = jnp.pad(b1, (0, hid_pad - hid))
    W2p = jnp.pad(W2, ((0, hid_pad - hid), (0, NANS_PAD - NANS)))
    b2p = jnp.pad(b2, (0, NANS_PAD - NANS), constant_values=-1e30)
    out = _fusion(fused, W1p, b1p[None, :], W2p, b2p[None, :])
    return out[:, :NANS]


# speedup vs baseline: 46.4603x; 1.1902x over previous
import functools
import math

import jax
import jax.numpy as jnp
from jax import lax
from jax.experimental import pallas as pl
from jax.experimental.pallas import tpu as pltpu
from jax.experimental.pallas import tpu_sc as plsc

TXT_N = 2048
OBJ_N = 10000
NB = 16
NANS = 3129
NANS_PAD = 3200
NC = 2
NS = 16
NW = NC * NS

F32 = jnp.float32


def _dense_pre(x, W, a_s, a_d, br):
    n, di = x.shape
    do = W.shape[1]
    nb = n // br
    a2 = jnp.concatenate(
        [a_s[None, :], a_d[None, :], jnp.zeros((6, do), F32)], axis=0)

    def body(x_ref, w_ref, a_ref, h_ref, aux_ref):
        h = jnp.dot(x_ref[...], w_ref[...], preferred_element_type=F32)
        h_ref[...] = h
        al = jnp.sum(h * a_ref[0, :][None, :], axis=1)
        ar = jnp.sum(h * a_ref[1, :][None, :], axis=1)
        aux_ref[...] = jnp.concatenate(
            [al[:, None], ar[:, None], jnp.zeros((br, 6), F32)], axis=1)

    h, aux = pl.pallas_call(
        body,
        grid=(nb,),
        in_specs=[
            pl.BlockSpec((br, di), lambda i: (i, 0)),
            pl.BlockSpec((di, do), lambda i: (0, 0)),
            pl.BlockSpec((8, do), lambda i: (0, 0)),
        ],
        out_specs=[
            pl.BlockSpec((br, do), lambda i: (i, 0)),
            pl.BlockSpec((br, 8), lambda i: (i, 0)),
        ],
        out_shape=[
            jax.ShapeDtypeStruct((n, do), F32),
            jax.ShapeDtypeStruct((n, 8), F32),
        ],
    )(x, W, a2)
    return h, aux


def _gat_edges(h, al, ar, src, dst, *, n, D, CE, ZR):
    E = src.shape[0]
    EP = E // NW
    NIT = EP // CE
    RP = n // NS
    NZ = RP // ZR

    mesh = plsc.VectorSubcoreMesh(core_axis_name="c", subcore_axis_name="s",
                                  num_cores=NC, num_subcores=NS)

    @functools.partial(
        pl.kernel,
        out_type=[
            jax.ShapeDtypeStruct((NC, n, D), F32),
            jax.ShapeDtypeStruct((NW, n), F32),
        ],
        mesh=mesh,
        compiler_params=pltpu.CompilerParams(
            needs_layout_passes=False, use_tc_tiling_on_sc=False),
        scratch_types=[
            pltpu.VMEM((n,), F32),
            pltpu.VMEM((n,), F32),
            pltpu.VMEM((n,), F32),
            pltpu.VMEM((2, CE), jnp.int32),
            pltpu.VMEM((2, CE), jnp.int32),
            pltpu.VMEM((CE,), F32),
            pltpu.VMEM((2, CE, D), F32),
            pltpu.VMEM_SHARED((n, D), F32),
            pltpu.SemaphoreType.DMA,
            pltpu.SemaphoreType.DMA,
            pltpu.SemaphoreType.DMA,
        ],
    )
    def k(h_hbm, al_hbm, ar_hbm, src_hbm, dst_hbm, num_hbm, den_hbm,
          alv, arv, denv, srcv, dstv, exv, rows, acc, sem_i, sem_ga,
          sem_gb):
        c = lax.axis_index("c")
        s = lax.axis_index("s")
        w = c * NS + s
        zero16 = jnp.zeros((16,), F32)

        pltpu.sync_copy(al_hbm, alv)
        pltpu.sync_copy(ar_hbm, arv)

        def zden(i, carry):
            denv[pl.ds(i * 16, 16)] = zero16
            return carry
        lax.fori_loop(0, n // 16, zden, 0)

        def zzb(r, carry):
            for fg in range(D // 16):
                rows[0, r, pl.ds(fg * 16, 16)] = zero16
            return carry
        lax.fori_loop(0, ZR, zzb, 0)

        def zacc(i, carry):
            pltpu.sync_copy(rows.at[0, pl.ds(0, ZR)],
                            acc.at[pl.ds(s * RP + i * ZR, ZR)])
            return carry
        lax.fori_loop(0, NZ, zacc, 0)

        neg = jnp.full((16,), -3e38, F32)

        def mx(i, carry):
            ca, cr = carry
            ca = jnp.maximum(ca, alv[pl.ds(i * 16, 16)])
            cr = jnp.maximum(cr, arv[pl.ds(i * 16, 16)])
            return ca, cr
        ca, cr = lax.fori_loop(0, n // 16, mx, (neg, neg))
        lanes = lax.iota(jnp.int32, 16)
        for k in (8, 4, 2, 1):
            exv[pl.ds(0, 16)] = ca
            ca = jnp.maximum(ca, plsc.load_gather(exv, [lanes ^ k]))
            exv[pl.ds(0, 16)] = cr
            cr = jnp.maximum(cr, plsc.load_gather(exv, [lanes ^ k]))
        mv = jnp.maximum(ca + cr, jnp.zeros((16,), F32))

        plsc.subcore_barrier()

        def fetch_idx(it, buf):
            base = w * EP + it * CE
            pltpu.async_copy(src_hbm.at[pl.ds(base, CE)], srcv.at[buf], sem_i)
            pltpu.async_copy(dst_hbm.at[pl.ds(base, CE)], dstv.at[buf], sem_i)

        def wait_idx(buf):
            pltpu.make_async_copy(
                src_hbm.at[pl.ds(0, CE)], srcv.at[buf], sem_i).wait()
            pltpu.make_async_copy(
                src_hbm.at[pl.ds(0, CE)], dstv.at[buf], sem_i).wait()

        gsems = (sem_ga, sem_gb)

        def issue_gather(p):
            pltpu.async_copy(h_hbm.at[srcv.at[p]], rows.at[p], gsems[p])

        def wait_gather(p):
            pltpu.make_async_copy(
                h_hbm.at[srcv.at[p]], rows.at[p], gsems[p]).wait()

        def chunk(i, p):
            q = 1 - p

            @pl.when(i + 1 < NIT)
            def _():
                wait_idx(q)
                issue_gather(q)
            for g in range(CE // 16):
                si = srcv[p, pl.ds(g * 16, 16)]
                di = dstv[p, pl.ds(g * 16, 16)]
                t = plsc.load_gather(alv, [si]) + plsc.load_gather(arv, [di])
                e = jnp.where(t > 0.0, t, 0.2 * t)
                ex = jnp.exp(e - mv)
                exv[pl.ds(g * 16, 16)] = ex
                plsc.addupdate_scatter(denv, [di], ex)
            wait_gather(p)

            def scale(g, carry):
                exg = exv[pl.ds(g * 16, 16)]
                for l in range(16):
                    eidx = g * 16 + l
                    exb = jnp.full((16,), exg[l], F32)
                    for fg in range(D // 16):
                        rows[p, eidx, pl.ds(fg * 16, 16)] = (
                            rows[p, eidx, pl.ds(fg * 16, 16)] * exb)
                return carry
            lax.fori_loop(0, CE // 16, scale, 0)
            pltpu.sync_copy(rows.at[p], acc.at[dstv.at[p]], add=True)

            @pl.when(i + 2 < NIT)
            def _():
                fetch_idx(i + 2, p)

        fetch_idx(0, 0)
        wait_idx(0)
        issue_gather(0)
        fetch_idx(1, 1)

        def body2(j, carry):
            chunk(j * 2, 0)
            chunk(j * 2 + 1, 1)
            return carry
        lax.fori_loop(0, NIT // 2, body2, 0)
        if NIT % 2 == 1:
            chunk(NIT - 1, 0)

        plsc.subcore_barrier()

        pltpu.sync_copy(denv, den_hbm.at[w])
        pltpu.sync_copy(acc.at[pl.ds(s * RP, RP)],
                        num_hbm.at[c, pl.ds(s * RP, RP)])

    return k(h, al, ar, src, dst)


def _gat_post(num, den, b, br):
    _, n, D = num.shape
    b2 = b[None, :]
    den_t = den.T

    def body(num_ref, den_ref, b_ref, out_ref):
        tot = num_ref[0] + num_ref[1]
        dd = jnp.sum(den_ref[...], axis=1)
        out_ref[...] = (tot / jnp.maximum(dd, 1e-30)[:, None]
                        + b_ref[0, :][None, :])

    return pl.pallas_call(
        body,
        grid=(n // br,),
        in_specs=[
            pl.BlockSpec((NC, br, D), lambda i: (0, i, 0)),
            pl.BlockSpec((br, NW), lambda i: (i, 0)),
            pl.BlockSpec((1, D), lambda i: (0, 0)),
        ],
        out_specs=pl.BlockSpec((br, D), lambda i: (i, 0)),
        out_shape=jax.ShapeDtypeStruct((n, D), F32),
    )(num, den_t, b2)


def _matmul(a, b):
    n, k = a.shape
    d = b.shape[1]

    def body(a_ref, b_ref, o_ref):
        o_ref[...] = jnp.dot(a_ref[...], b_ref[...],
                             preferred_element_type=F32)

    return pl.pallas_call(
        body,
        out_shape=jax.ShapeDtypeStruct((n, d), F32),
    )(a, b)


def _biatt_a(F, Wk, Q, Sm, br):
    n, d = F.shape
    nt = Sm.shape[0]
    nb = n // br
    scale = 1.0 / math.sqrt(float(d))

    def body(f_ref, wk_ref, q_ref, sm_ref, out_ref, cm_ref):
        i = pl.program_id(0)
        K = jnp.dot(f_ref[...], wk_ref[...], preferred_element_type=F32)
        S = lax.dot_general(K, q_ref[...], (((1,), (1,)), ((), ())),
                            preferred_element_type=F32) * scale
        rm = jnp.max(S, axis=1)
        P = jnp.exp(S - rm[:, None])
        rs = jnp.sum(P, axis=1)
        out_ref[...] = f_ref[...] + (
            jnp.dot(P, sm_ref[...], preferred_element_type=F32)
            / rs[:, None])
        bm = jnp.max(S, axis=0)[None, :]

        @pl.when(i == 0)
        def _():
            cm_ref[...] = bm

        @pl.when(i > 0)
        def _():
            cm_ref[...] = jnp.maximum(cm_ref[...], bm)

    return pl.pallas_call(
        body,
        grid=(nb,),
        in_specs=[
            pl.BlockSpec((br, d), lambda i: (i, 0)),
            pl.BlockSpec((d, d), lambda i: (0, 0)),
            pl.BlockSpec((nt, d), lambda i: (0, 0)),
            pl.BlockSpec((nt, d), lambda i: (0, 0)),
        ],
        out_specs=[
            pl.BlockSpec((br, d), lambda i: (i, 0)),
            pl.BlockSpec((1, nt), lambda i: (0, 0)),
        ],
        out_shape=[
            jax.ShapeDtypeStruct((n, d), F32),
            jax.ShapeDtypeStruct((1, nt), F32),
        ],
    )(F, Wk, Q, Sm)


def _biatt_b(F, Wk, Q, Sm, colmax, br):
    n, d = F.shape
    nt = Sm.shape[0]
    nb = n // br
    scale = 1.0 / math.sqrt(float(d))

    def body(f_ref, wk_ref, q_ref, sm_ref, cm_ref, out_ref, nacc, cs):
        i = pl.program_id(0)

        @pl.when(i == 0)
        def _():
            nacc[...] = jnp.zeros_like(nacc)
            cs[...] = jnp.zeros_like(cs)

        K = jnp.dot(f_ref[...], wk_ref[...], preferred_element_type=F32)
        S = lax.dot_general(K, q_ref[...], (((1,), (1,)), ((), ())),
                            preferred_element_type=F32) * scale
        Eexp = jnp.exp(S - cm_ref[...])
        cs[...] += jnp.sum(Eexp, axis=0)[None, :]
        nacc[...] += lax.dot_general(Eexp, f_ref[...],
                                     (((0,), (0,)), ((), ())),
                                     preferred_element_type=F32)

        @pl.when(i == nb - 1)
        def _():
            out_ref[...] = sm_ref[...] + nacc[...] / cs[0, :][:, None]

    return pl.pallas_call(
        body,
        grid=(nb,),
        in_specs=[
            pl.BlockSpec((br, d), lambda i: (i, 0)),
            pl.BlockSpec((d, d), lambda i: (0, 0)),
            pl.BlockSpec((nt, d), lambda i: (0, 0)),
            pl.BlockSpec((nt, d), lambda i: (0, 0)),
            pl.BlockSpec((1, nt), lambda i: (0, 0)),
        ],
        out_specs=pl.BlockSpec((nt, d), lambda i: (0, 0)),
        out_shape=jax.ShapeDtypeStruct((nt, d), F32),
        scratch_shapes=[
            pltpu.VMEM((nt, d), F32),
            pltpu.VMEM((1, nt), F32),
        ],
    )(F, Wk, Q, Sm, colmax)


def _pool(x, batch8, br):
    n, D = x.shape
    nb = n // br

    def body(x_ref, b_ref, out_ref, sums, cnts):
        i = pl.program_id(0)

        @pl.when(i == 0)
        def _():
            sums[...] = jnp.zeros_like(sums)
            cnts[...] = jnp.zeros_like(cnts)

        ids = b_ref[:, 0]
        oh = (lax.broadcasted_iota(jnp.int32, (NB, br), 0)
              == ids[None, :]).astype(F32)
        sums[...] += jnp.dot(oh, x_ref[...], preferred_element_type=F32)
        cnts[...] += jnp.broadcast_to(jnp.sum(oh, axis=1)[:, None], (NB, D))

        @pl.when(i == nb - 1)
        def _():
            out_ref[...] = sums[...] / jnp.maximum(cnts[...], 1.0)

    return pl.pallas_call(
        body,
        grid=(nb,),
        in_specs=[
            pl.BlockSpec((br, D), lambda i: (i, 0)),
            pl.BlockSpec((br, 8), lambda i: (i, 0)),
        ],
        out_specs=pl.BlockSpec((NB, D), lambda i: (0, 0)),
        out_shape=jax.ShapeDtypeStruct((NB, D), F32),
        scratch_shapes=[
            pltpu.VMEM((NB, D), F32),
            pltpu.VMEM((NB, D), F32),
        ],
    )(x, batch8)


def _fusion(fused, W1, b1, W2p, b2p):
    in_dim, hid = W1.shape
    KB = 256
    nkb = hid // KB

    def body(f_ref, w1_ref, b1_ref, w2_ref, b2_ref, out_ref, acc):
        i = pl.program_id(0)

        @pl.when(i == 0)
        def _():
            acc[...] = jnp.broadcast_to(b2_ref[0, :][None, :], acc.shape)

        h1 = (jnp.dot(f_ref[...], w1_ref[...], preferred_element_type=F32)
              + b1_ref[0, :][None, :])
        acc[...] += jnp.dot(h1, w2_ref[...], preferred_element_type=F32)

        @pl.when(i == nkb - 1)
        def _():
            logits = acc[...]
            z = logits - jnp.max(logits, axis=1, keepdims=True)
            out_ref[...] = z - jnp.log(
                jnp.sum(jnp.exp(z), axis=1, keepdims=True))

    return pl.pallas_call(
        body,
        grid=(nkb,),
        in_specs=[
            pl.BlockSpec((NB, in_dim), lambda i: (0, 0)),
            pl.BlockSpec((in_dim, KB), lambda i: (0, i)),
            pl.BlockSpec((1, KB), lambda i: (0, i)),
            pl.BlockSpec((KB, NANS_PAD), lambda i: (i, 0)),
            pl.BlockSpec((1, NANS_PAD), lambda i: (0, 0)),
        ],
        out_specs=pl.BlockSpec((NB, NANS_PAD), lambda i: (0, 0)),
        out_shape=jax.ShapeDtypeStruct((NB, NANS_PAD), F32),
        scratch_shapes=[pltpu.VMEM((NB, NANS_PAD), F32)],
    )(fused, W1, b1, W2p, b2p)


def _gat_layer(x, src, dst, p, *, n, br, CE, ZR, pad_to=None):
    W, a_s, a_d, b = p
    if pad_to is not None and W.shape[1] < pad_to:
        extra = pad_to - W.shape[1]
        W = jnp.pad(W, ((0, 0), (0, extra)))
        a_s = jnp.pad(a_s, (0, extra))
        a_d = jnp.pad(a_d, (0, extra))
        b = jnp.pad(b, (0, extra))
    D = W.shape[1]
    h, aux = _dense_pre(x, W, a_s, a_d, br)
    num, den = _gat_edges(h, aux[:, 0], aux[:, 1], src, dst,
                          n=n, D=D, CE=CE, ZR=ZR)
    return _gat_post(num, den, b, br)


def kernel(params, word_syn_x, txt_syn_edge_index, txt_sem_x,
           txt_sem_edge_index, objects, obj_edge_index, obj_pos_x,
           txt_batch, obj_batch):
    syn, sem, objf, objp = word_syn_x, txt_sem_x, objects, obj_pos_x
    ts_src, ts_dst = txt_syn_edge_index[0], txt_syn_edge_index[1]
    tm_src, tm_dst = txt_sem_edge_index[0], txt_sem_edge_index[1]
    ob_src, ob_dst = obj_edge_index[0], obj_edge_index[1]

    txt_kw = dict(n=TXT_N, br=512, CE=64, ZR=64)
    obj_kw = dict(n=OBJ_N, br=1000, CE=80, ZR=25)

    for lvl in range(3):
        syn = _gat_layer(syn, ts_src, ts_dst, params['syn'][lvl], **txt_kw)
        sem = _gat_layer(sem, tm_src, tm_dst, params['sem'][lvl], **txt_kw)
        objf = _gat_layer(objf, ob_src, ob_dst, params['objf'][lvl],
                          **obj_kw)
        objp = _gat_layer(objp, ob_src, ob_dst, params['objp'][lvl],
                          pad_to=16, **obj_kw)
        Wk, Wq = params['biatt'][lvl]
        Q = _matmul(sem, Wq)
        objf_new, colmax = _biatt_a(objf, Wk, Q, sem, br=1000)
        sem = _biatt_b(objf, Wk, Q, sem, colmax, br=1000)
        objf = objf_new

    txt_b8 = jnp.broadcast_to(txt_batch[:, None], (TXT_N, 8))
    obj_b8 = jnp.broadcast_to(obj_batch[:, None], (OBJ_N, 8))
    syn_p = _pool(syn, txt_b8, br=512)
    sem_p = _pool(sem, txt_b8, br=512)
    objf_p = _pool(objf, obj_b8, br=1000)
    objp_p = _pool(objp, obj_b8, br=1000)[:, :8]

    fused = jnp.concatenate([syn_p, objf_p, sem_p, objp_p], axis=1)
    W1, b1, W2, b2 = params['fusion']
    hid = W1.shape[1]
    hid_pad = ((hid + 255) // 256) * 256
    W1p = jnp.pad(W1, ((0, 0), (0, hid_pad - hid)))
    b1p = jnp.pad(b1, (0, hid_pad - hid))
    W2p = jnp.pad(W2, ((0, hid_pad - hid), (0, NANS_PAD - NANS)))
    b2p = jnp.pad(b2, (0, NANS_PAD - NANS), constant_values=-1e30)
    out = _fusion(fused, W1p, b1p[None, :], W2p, b2p[None, :])
    return out[:, :NANS]


# speedup vs baseline: 47.1505x; 1.0149x over previous
import functools
import math

import jax
import jax.numpy as jnp
from jax import lax
from jax.experimental import pallas as pl
from jax.experimental.pallas import tpu as pltpu
from jax.experimental.pallas import tpu_sc as plsc

TXT_N = 2048
OBJ_N = 10000
NB = 16
NANS = 3129
NANS_PAD = 3200
NC = 2
NS = 16
NW = NC * NS

F32 = jnp.float32


def _dense_pre(x, W, a_s, a_d, br):
    n, di = x.shape
    do = W.shape[1]
    nb = n // br
    a2 = jnp.concatenate(
        [a_s[None, :], a_d[None, :], jnp.zeros((6, do), F32)], axis=0)

    def body(x_ref, w_ref, a_ref, h_ref, aux_ref):
        h = jnp.dot(x_ref[...], w_ref[...], preferred_element_type=F32)
        h_ref[...] = h
        al = jnp.sum(h * a_ref[0, :][None, :], axis=1)
        ar = jnp.sum(h * a_ref[1, :][None, :], axis=1)
        aux_ref[...] = jnp.concatenate(
            [al[:, None], ar[:, None], jnp.zeros((br, 6), F32)], axis=1)

    h, aux = pl.pallas_call(
        body,
        grid=(nb,),
        in_specs=[
            pl.BlockSpec((br, di), lambda i: (i, 0)),
            pl.BlockSpec((di, do), lambda i: (0, 0)),
            pl.BlockSpec((8, do), lambda i: (0, 0)),
        ],
        out_specs=[
            pl.BlockSpec((br, do), lambda i: (i, 0)),
            pl.BlockSpec((br, 8), lambda i: (i, 0)),
        ],
        out_shape=[
            jax.ShapeDtypeStruct((n, do), F32),
            jax.ShapeDtypeStruct((n, 8), F32),
        ],
    )(x, W, a2)
    return h, aux


def _gat_edges(h, al, ar, src, dst, *, n, D, CE, ZR):
    E = src.shape[0]
    EP = E // NW
    NIT = EP // CE
    RP = n // NS
    NZ = RP // ZR

    mesh = plsc.VectorSubcoreMesh(core_axis_name="c", subcore_axis_name="s",
                                  num_cores=NC, num_subcores=NS)

    @functools.partial(
        pl.kernel,
        out_type=[
            jax.ShapeDtypeStruct((NC, n, D), F32),
            jax.ShapeDtypeStruct((NW, n), F32),
        ],
        mesh=mesh,
        compiler_params=pltpu.CompilerParams(
            needs_layout_passes=False, use_tc_tiling_on_sc=False),
        scratch_types=[
            pltpu.VMEM((n,), F32),
            pltpu.VMEM((n,), F32),
            pltpu.VMEM((n,), F32),
            pltpu.VMEM((2, CE), jnp.int32),
            pltpu.VMEM((2, CE), jnp.int32),
            pltpu.VMEM((CE,), F32),
            pltpu.VMEM((2, CE, D), F32),
            pltpu.VMEM_SHARED((n, D), F32),
            pltpu.SemaphoreType.DMA,
            pltpu.SemaphoreType.DMA,
            pltpu.SemaphoreType.DMA,
        ],
    )
    def k(h_hbm, al_hbm, ar_hbm, src_hbm, dst_hbm, num_hbm, den_hbm,
          alv, arv, denv, srcv, dstv, exv, rows, acc, sem_i, sem_ga,
          sem_gb):
        c = lax.axis_index("c")
        s = lax.axis_index("s")
        w = c * NS + s
        zero16 = jnp.zeros((16,), F32)

        pltpu.sync_copy(al_hbm, alv)
        pltpu.sync_copy(ar_hbm, arv)

        def zzb(r, carry):
            for fg in range(D // 16):
                rows[0, r, pl.ds(fg * 16, 16)] = zero16
            return carry
        lax.fori_loop(0, ZR, zzb, 0)

        def zacc(i, carry):
            pltpu.async_copy(rows.at[0, pl.ds(0, ZR)],
                             acc.at[pl.ds(s * RP + i * ZR, ZR)], sem_ga)
            return carry
        lax.fori_loop(0, NZ, zacc, 0)

        def zden(i, carry):
            denv[pl.ds(i * 16, 16)] = zero16
            return carry
        lax.fori_loop(0, n // 16, zden, 0)

        neg = jnp.full((16,), -3e38, F32)

        def mx(i, carry):
            ca, cr = carry
            ca = jnp.maximum(ca, alv[pl.ds(i * 16, 16)])
            cr = jnp.maximum(cr, arv[pl.ds(i * 16, 16)])
            return ca, cr
        ca, cr = lax.fori_loop(0, n // 16, mx, (neg, neg))
        lanes = lax.iota(jnp.int32, 16)
        for k in (8, 4, 2, 1):
            exv[pl.ds(0, 16)] = ca
            ca = jnp.maximum(ca, plsc.load_gather(exv, [lanes ^ k]))
            exv[pl.ds(0, 16)] = cr
            cr = jnp.maximum(cr, plsc.load_gather(exv, [lanes ^ k]))
        mv = jnp.maximum(ca + cr, jnp.zeros((16,), F32))

        def zdrain(i, carry):
            pltpu.make_async_copy(
                rows.at[0, pl.ds(0, ZR)],
                acc.at[pl.ds(s * RP + i * ZR, ZR)], sem_ga).wait()
            return carry
        lax.fori_loop(0, NZ, zdrain, 0)

        plsc.subcore_barrier()

        def fetch_idx(it, buf):
            base = w * EP + it * CE
            pltpu.async_copy(src_hbm.at[pl.ds(base, CE)], srcv.at[buf], sem_i)
            pltpu.async_copy(dst_hbm.at[pl.ds(base, CE)], dstv.at[buf], sem_i)

        def wait_idx(buf):
            pltpu.make_async_copy(
                src_hbm.at[pl.ds(0, CE)], srcv.at[buf], sem_i).wait()
            pltpu.make_async_copy(
                src_hbm.at[pl.ds(0, CE)], dstv.at[buf], sem_i).wait()

        gsems = (sem_ga, sem_gb)

        def issue_gather(p):
            pltpu.async_copy(h_hbm.at[srcv.at[p]], rows.at[p], gsems[p])

        def wait_gather(p):
            pltpu.make_async_copy(
                h_hbm.at[srcv.at[p]], rows.at[p], gsems[p]).wait()

        def chunk(i, p):
            q = 1 - p

            @pl.when(i + 1 < NIT)
            def _():
                wait_idx(q)
                issue_gather(q)
            for g in range(CE // 16):
                si = srcv[p, pl.ds(g * 16, 16)]
                di = dstv[p, pl.ds(g * 16, 16)]
                t = plsc.load_gather(alv, [si]) + plsc.load_gather(arv, [di])
                e = jnp.where(t > 0.0, t, 0.2 * t)
                ex = jnp.exp(e - mv)
                exv[pl.ds(g * 16, 16)] = ex
                plsc.addupdate_scatter(denv, [di], ex)
            wait_gather(p)

            def scale(g, carry):
                exg = exv[pl.ds(g * 16, 16)]
                for l in range(16):
                    eidx = g * 16 + l
                    exb = jnp.full((16,), exg[l], F32)
                    for fg in range(D // 16):
                        rows[p, eidx, pl.ds(fg * 16, 16)] = (
                            rows[p, eidx, pl.ds(fg * 16, 16)] * exb)
                return carry
            lax.fori_loop(0, CE // 16, scale, 0)
            pltpu.sync_copy(rows.at[p], acc.at[dstv.at[p]], add=True)

            @pl.when(i + 2 < NIT)
            def _():
                fetch_idx(i + 2, p)

        fetch_idx(0, 0)
        wait_idx(0)
        issue_gather(0)
        fetch_idx(1, 1)

        def body2(j, carry):
            chunk(j * 2, 0)
            chunk(j * 2 + 1, 1)
            return carry
        lax.fori_loop(0, NIT // 2, body2, 0)
        if NIT % 2 == 1:
            chunk(NIT - 1, 0)

        plsc.subcore_barrier()

        pltpu.sync_copy(denv, den_hbm.at[w])
        pltpu.sync_copy(acc.at[pl.ds(s * RP, RP)],
                        num_hbm.at[c, pl.ds(s * RP, RP)])

    return k(h, al, ar, src, dst)


def _gat_post(num, den, b, br):
    _, n, D = num.shape
    b2 = b[None, :]
    den_t = den.T

    def body(num_ref, den_ref, b_ref, out_ref):
        tot = num_ref[0] + num_ref[1]
        dd = jnp.sum(den_ref[...], axis=1)
        out_ref[...] = (tot / jnp.maximum(dd, 1e-30)[:, None]
                        + b_ref[0, :][None, :])

    return pl.pallas_call(
        body,
        grid=(n // br,),
        in_specs=[
            pl.BlockSpec((NC, br, D), lambda i: (0, i, 0)),
            pl.BlockSpec((br, NW), lambda i: (i, 0)),
            pl.BlockSpec((1, D), lambda i: (0, 0)),
        ],
        out_specs=pl.BlockSpec((br, D), lambda i: (i, 0)),
        out_shape=jax.ShapeDtypeStruct((n, D), F32),
    )(num, den_t, b2)


def _matmul(a, b):
    n, k = a.shape
    d = b.shape[1]

    def body(a_ref, b_ref, o_ref):
        o_ref[...] = jnp.dot(a_ref[...], b_ref[...],
                             preferred_element_type=F32)

    return pl.pallas_call(
        body,
        out_shape=jax.ShapeDtypeStruct((n, d), F32),
    )(a, b)


def _biatt_a(F, Wk, Q, Sm, br):
    n, d = F.shape
    nt = Sm.shape[0]
    nb = n // br
    scale = 1.0 / math.sqrt(float(d))

    def body(f_ref, wk_ref, q_ref, sm_ref, out_ref, cm_ref):
        i = pl.program_id(0)
        K = jnp.dot(f_ref[...], wk_ref[...], preferred_element_type=F32)
        S = lax.dot_general(K, q_ref[...], (((1,), (1,)), ((), ())),
                            preferred_element_type=F32) * scale
        rm = jnp.max(S, axis=1)
        P = jnp.exp(S - rm[:, None])
        rs = jnp.sum(P, axis=1)
        out_ref[...] = f_ref[...] + (
            jnp.dot(P, sm_ref[...], preferred_element_type=F32)
            / rs[:, None])
        bm = jnp.max(S, axis=0)[None, :]

        @pl.when(i == 0)
        def _():
            cm_ref[...] = bm

        @pl.when(i > 0)
        def _():
            cm_ref[...] = jnp.maximum(cm_ref[...], bm)

    return pl.pallas_call(
        body,
        grid=(nb,),
        in_specs=[
            pl.BlockSpec((br, d), lambda i: (i, 0)),
            pl.BlockSpec((d, d), lambda i: (0, 0)),
            pl.BlockSpec((nt, d), lambda i: (0, 0)),
            pl.BlockSpec((nt, d), lambda i: (0, 0)),
        ],
        out_specs=[
            pl.BlockSpec((br, d), lambda i: (i, 0)),
            pl.BlockSpec((1, nt), lambda i: (0, 0)),
        ],
        out_shape=[
            jax.ShapeDtypeStruct((n, d), F32),
            jax.ShapeDtypeStruct((1, nt), F32),
        ],
    )(F, Wk, Q, Sm)


def _biatt_b(F, Wk, Q, Sm, colmax, br):
    n, d = F.shape
    nt = Sm.shape[0]
    nb = n // br
    scale = 1.0 / math.sqrt(float(d))

    def body(f_ref, wk_ref, q_ref, sm_ref, cm_ref, out_ref, nacc, cs):
        i = pl.program_id(0)

        @pl.when(i == 0)
        def _():
            nacc[...] = jnp.zeros_like(nacc)
            cs[...] = jnp.zeros_like(cs)

        K = jnp.dot(f_ref[...], wk_ref[...], preferred_element_type=F32)
        S = lax.dot_general(K, q_ref[...], (((1,), (1,)), ((), ())),
                            preferred_element_type=F32) * scale
        Eexp = jnp.exp(S - cm_ref[...])
        cs[...] += jnp.sum(Eexp, axis=0)[None, :]
        nacc[...] += lax.dot_general(Eexp, f_ref[...],
                                     (((0,), (0,)), ((), ())),
                                     preferred_element_type=F32)

        @pl.when(i == nb - 1)
        def _():
            out_ref[...] = sm_ref[...] + nacc[...] / cs[0, :][:, None]

    return pl.pallas_call(
        body,
        grid=(nb,),
        in_specs=[
            pl.BlockSpec((br, d), lambda i: (i, 0)),
            pl.BlockSpec((d, d), lambda i: (0, 0)),
            pl.BlockSpec((nt, d), lambda i: (0, 0)),
            pl.BlockSpec((nt, d), lambda i: (0, 0)),
            pl.BlockSpec((1, nt), lambda i: (0, 0)),
        ],
        out_specs=pl.BlockSpec((nt, d), lambda i: (0, 0)),
        out_shape=jax.ShapeDtypeStruct((nt, d), F32),
        scratch_shapes=[
            pltpu.VMEM((nt, d), F32),
            pltpu.VMEM((1, nt), F32),
        ],
    )(F, Wk, Q, Sm, colmax)


def _pool(x, batch8, br):
    n, D = x.shape
    nb = n // br

    def body(x_ref, b_ref, out_ref, sums, cnts):
        i = pl.program_id(0)

        @pl.when(i == 0)
        def _():
            sums[...] = jnp.zeros_like(sums)
            cnts[...] = jnp.zeros_like(cnts)

        ids = b_ref[:, 0]
        oh = (lax.broadcasted_iota(jnp.int32, (NB, br), 0)
              == ids[None, :]).astype(F32)
        sums[...] += jnp.dot(oh, x_ref[...], preferred_element_type=F32)
        cnts[...] += jnp.broadcast_to(jnp.sum(oh, axis=1)[:, None], (NB, D))

        @pl.when(i == nb - 1)
        def _():
            out_ref[...] = sums[...] / jnp.maximum(cnts[...], 1.0)

    return pl.pallas_call(
        body,
        grid=(nb,),
        in_specs=[
            pl.BlockSpec((br, D), lambda i: (i, 0)),
            pl.BlockSpec((br, 8), lambda i: (i, 0)),
        ],
        out_specs=pl.BlockSpec((NB, D), lambda i: (0, 0)),
        out_shape=jax.ShapeDtypeStruct((NB, D), F32),
        scratch_shapes=[
            pltpu.VMEM((NB, D), F32),
            pltpu.VMEM((NB, D), F32),
        ],
    )(x, batch8)


def _fusion(fused, W1, b1, W2p, b2p):
    in_dim, hid = W1.shape
    KB = 256
    nkb = hid // KB

    def body(f_ref, w1_ref, b1_ref, w2_ref, b2_ref, out_ref, acc):
        i = pl.program_id(0)

        @pl.when(i == 0)
        def _():
            acc[...] = jnp.broadcast_to(b2_ref[0, :][None, :], acc.shape)

        h1 = (jnp.dot(f_ref[...], w1_ref[...], preferred_element_type=F32)
              + b1_ref[0, :][None, :])
        acc[...] += jnp.dot(h1, w2_ref[...], preferred_element_type=F32)

        @pl.when(i == nkb - 1)
        def _():
            logits = acc[...]
            z = logits - jnp.max(logits, axis=1, keepdims=True)
            out_ref[...] = z - jnp.log(
                jnp.sum(jnp.exp(z), axis=1, keepdims=True))

    return pl.pallas_call(
        body,
        grid=(nkb,),
        in_specs=[
            pl.BlockSpec((NB, in_dim), lambda i: (0, 0)),
            pl.BlockSpec((in_dim, KB), lambda i: (0, i)),
            pl.BlockSpec((1, KB), lambda i: (0, i)),
            pl.BlockSpec((KB, NANS_PAD), lambda i: (i, 0)),
            pl.BlockSpec((1, NANS_PAD), lambda i: (0, 0)),
        ],
        out_specs=pl.BlockSpec((NB, NANS_PAD), lambda i: (0, 0)),
        out_shape=jax.ShapeDtypeStruct((NB, NANS_PAD), F32),
        scratch_shapes=[pltpu.VMEM((NB, NANS_PAD), F32)],
    )(fused, W1, b1, W2p, b2p)


def _gat_layer(x, src, dst, p, *, n, br, CE, ZR, pad_to=None):
    W, a_s, a_d, b = p
    if pad_to is not None and W.shape[1] < pad_to:
        extra = pad_to - W.shape[1]
        W = jnp.pad(W, ((0, 0), (0, extra)))
        a_s = jnp.pad(a_s, (0, extra))
        a_d = jnp.pad(a_d, (0, extra))
        b = jnp.pad(b, (0, extra))
    D = W.shape[1]
    h, aux = _dense_pre(x, W, a_s, a_d, br)
    num, den = _gat_edges(h, aux[:, 0], aux[:, 1], src, dst,
                          n=n, D=D, CE=CE, ZR=ZR)
    return _gat_post(num, den, b, br)


def kernel(params, word_syn_x, txt_syn_edge_index, txt_sem_x,
           txt_sem_edge_index, objects, obj_edge_index, obj_pos_x,
           txt_batch, obj_batch):
    syn, sem, objf, objp = word_syn_x, txt_sem_x, objects, obj_pos_x
    ts_src, ts_dst = txt_syn_edge_index[0], txt_syn_edge_index[1]
    tm_src, tm_dst = txt_sem_edge_index[0], txt_sem_edge_index[1]
    ob_src, ob_dst = obj_edge_index[0], obj_edge_index[1]

    txt_kw = dict(n=TXT_N, br=512, CE=64, ZR=64)
    obj_kw = dict(n=OBJ_N, br=1000, CE=80, ZR=25)

    for lvl in range(3):
        syn = _gat_layer(syn, ts_src, ts_dst, params['syn'][lvl], **txt_kw)
        sem = _gat_layer(sem, tm_src, tm_dst, params['sem'][lvl], **txt_kw)
        objf = _gat_layer(objf, ob_src, ob_dst, params['objf'][lvl],
                          **obj_kw)
        objp = _gat_layer(objp, ob_src, ob_dst, params['objp'][lvl],
                          pad_to=16, **obj_kw)
        Wk, Wq = params['biatt'][lvl]
        Q = _matmul(sem, Wq)
        objf_new, colmax = _biatt_a(objf, Wk, Q, sem, br=1000)
        sem = _biatt_b(objf, Wk, Q, sem, colmax, br=1000)
        objf = objf_new

    txt_b8 = jnp.broadcast_to(txt_batch[:, None], (TXT_N, 8))
    obj_b8 = jnp.broadcast_to(obj_batch[:, None], (OBJ_N, 8))
    syn_p = _pool(syn, txt_b8, br=512)
    sem_p = _pool(sem, txt_b8, br=512)
    objf_p = _pool(objf, obj_b8, br=1000)
    objp_p = _pool(objp, obj_b8, br=1000)[:, :8]

    fused = jnp.concatenate([syn_p, objf_p, sem_p, objp_p], axis=1)
    W1, b1, W2, b2 = params['fusion']
    hid = W1.shape[1]
    hid_pad = ((hid + 255) // 256) * 256
    W1p = jnp.pad(W1, ((0, 0), (0, hid_pad - hid)))
    b1p = jnp.pad(b1, (0, hid_pad - hid))
    W2p = jnp.pad(W2, ((0, hid_pad - hid), (0, NANS_PAD - NANS)))
    b2p = jnp.pad(b2, (0, NANS_PAD - NANS), constant_values=-1e30)
    out = _fusion(fused, W1p, b1p[None, :], W2p, b2p[None, :])
    return out[:, :NANS]


# speedup vs baseline: 47.4272x; 1.0059x over previous
import functools
import math

import jax
import jax.numpy as jnp
from jax import lax
from jax.experimental import pallas as pl
from jax.experimental.pallas import tpu as pltpu
from jax.experimental.pallas import tpu_sc as plsc

TXT_N = 2048
OBJ_N = 10000
NB = 16
NANS = 3129
NANS_PAD = 3200
NC = 2
NS = 16
NW = NC * NS

F32 = jnp.float32


def _dense_pre(x, W, a_s, a_d, br):
    n, di = x.shape
    do = W.shape[1]
    nb = n // br
    a2 = jnp.concatenate(
        [a_s[None, :], a_d[None, :], jnp.zeros((6, do), F32)], axis=0)

    def body(x_ref, w_ref, a_ref, h_ref, aux_ref):
        h = jnp.dot(x_ref[...], w_ref[...], preferred_element_type=F32)
        h_ref[...] = h
        al = jnp.sum(h * a_ref[0, :][None, :], axis=1)
        ar = jnp.sum(h * a_ref[1, :][None, :], axis=1)
        aux_ref[...] = jnp.concatenate(
            [al[:, None], ar[:, None], jnp.zeros((br, 6), F32)], axis=1)

    h, aux = pl.pallas_call(
        body,
        grid=(nb,),
        in_specs=[
            pl.BlockSpec((br, di), lambda i: (i, 0)),
            pl.BlockSpec((di, do), lambda i: (0, 0)),
            pl.BlockSpec((8, do), lambda i: (0, 0)),
        ],
        out_specs=[
            pl.BlockSpec((br, do), lambda i: (i, 0)),
            pl.BlockSpec((br, 8), lambda i: (i, 0)),
        ],
        out_shape=[
            jax.ShapeDtypeStruct((n, do), F32),
            jax.ShapeDtypeStruct((n, 8), F32),
        ],
    )(x, W, a2)
    return h, aux


def _gat_edges(h, al, ar, src, dst, *, n, D, CE, ZR):
    E = src.shape[0]
    EP = E // NW
    NIT = EP // CE
    RP = n // NS
    NZ = RP // ZR

    mesh = plsc.VectorSubcoreMesh(core_axis_name="c", subcore_axis_name="s",
                                  num_cores=NC, num_subcores=NS)

    @functools.partial(
        pl.kernel,
        out_type=[
            jax.ShapeDtypeStruct((NC, n, D), F32),
            jax.ShapeDtypeStruct((NW, n), F32),
        ],
        mesh=mesh,
        compiler_params=pltpu.CompilerParams(
            needs_layout_passes=False, use_tc_tiling_on_sc=False),
        scratch_types=[
            pltpu.VMEM((n,), F32),
            pltpu.VMEM((n,), F32),
            pltpu.VMEM((n,), F32),
            pltpu.VMEM((2, CE), jnp.int32),
            pltpu.VMEM((2, CE), jnp.int32),
            pltpu.VMEM((CE,), F32),
            pltpu.VMEM((2, CE, D), F32),
            pltpu.VMEM_SHARED((n, D), F32),
            pltpu.SemaphoreType.DMA,
            pltpu.SemaphoreType.DMA,
            pltpu.SemaphoreType.DMA,
        ],
    )
    def k(h_hbm, al_hbm, ar_hbm, src_hbm, dst_hbm, num_hbm, den_hbm,
          alv, arv, denv, srcv, dstv, exv, rows, acc, sem_i, sem_ga,
          sem_gb):
        c = lax.axis_index("c")
        s = lax.axis_index("s")
        w = c * NS + s
        zero16 = jnp.zeros((16,), F32)

        pltpu.sync_copy(al_hbm, alv)
        pltpu.sync_copy(ar_hbm, arv)

        def zzb(r, carry):
            for fg in range(D // 16):
                rows[0, r, pl.ds(fg * 16, 16)] = zero16
            return carry
        lax.fori_loop(0, ZR, zzb, 0)

        def zacc(i, carry):
            pltpu.async_copy(rows.at[0, pl.ds(0, ZR)],
                             acc.at[pl.ds(s * RP + i * ZR, ZR)], sem_ga)
            return carry
        lax.fori_loop(0, NZ, zacc, 0)

        def zden(i, carry):
            denv[pl.ds(i * 16, 16)] = zero16
            return carry
        lax.fori_loop(0, n // 16, zden, 0)

        neg = jnp.full((16,), -3e38, F32)

        def mx(i, carry):
            ca, cr = carry
            ca = jnp.maximum(ca, alv[pl.ds(i * 16, 16)])
            cr = jnp.maximum(cr, arv[pl.ds(i * 16, 16)])
            return ca, cr
        ca, cr = lax.fori_loop(0, n // 16, mx, (neg, neg))
        lanes = lax.iota(jnp.int32, 16)
        for k in (8, 4, 2, 1):
            exv[pl.ds(0, 16)] = ca
            ca = jnp.maximum(ca, plsc.load_gather(exv, [lanes ^ k]))
            exv[pl.ds(0, 16)] = cr
            cr = jnp.maximum(cr, plsc.load_gather(exv, [lanes ^ k]))
        mv = jnp.maximum(ca + cr, jnp.zeros((16,), F32))

        def zdrain(i, carry):
            pltpu.make_async_copy(
                rows.at[0, pl.ds(0, ZR)],
                acc.at[pl.ds(s * RP + i * ZR, ZR)], sem_ga).wait()
            return carry
        lax.fori_loop(0, NZ, zdrain, 0)

        plsc.subcore_barrier()

        def fetch_idx(it, buf):
            base = w * EP + it * CE
            pltpu.async_copy(src_hbm.at[pl.ds(base, CE)], srcv.at[buf], sem_i)
            pltpu.async_copy(dst_hbm.at[pl.ds(base, CE)], dstv.at[buf], sem_i)

        def wait_idx(buf):
            pltpu.make_async_copy(
                src_hbm.at[pl.ds(0, CE)], srcv.at[buf], sem_i).wait()
            pltpu.make_async_copy(
                src_hbm.at[pl.ds(0, CE)], dstv.at[buf], sem_i).wait()

        gsems = (sem_ga, sem_gb)

        def issue_gather(p):
            pltpu.async_copy(h_hbm.at[srcv.at[p]], rows.at[p], gsems[p])

        def wait_gather(p):
            pltpu.make_async_copy(
                h_hbm.at[srcv.at[p]], rows.at[p], gsems[p]).wait()

        def chunk(i, p):
            q = 1 - p

            @pl.when(i + 1 < NIT)
            def _():
                wait_idx(q)
                issue_gather(q)
            for g in range(CE // 16):
                si = srcv[p, pl.ds(g * 16, 16)]
                di = dstv[p, pl.ds(g * 16, 16)]
                t = plsc.load_gather(alv, [si]) + plsc.load_gather(arv, [di])
                e = jnp.where(t > 0.0, t, 0.2 * t)
                ex = jnp.exp(e - mv)
                exv[pl.ds(g * 16, 16)] = ex
                plsc.addupdate_scatter(denv, [di], ex)
            wait_gather(p)

            def scale(g, carry):
                exg = exv[pl.ds(g * 16, 16)]
                for l in range(16):
                    eidx = g * 16 + l
                    exb = jnp.full((16,), exg[l], F32)
                    for fg in range(D // 16):
                        rows[p, eidx, pl.ds(fg * 16, 16)] = (
                            rows[p, eidx, pl.ds(fg * 16, 16)] * exb)
                return carry
            lax.fori_loop(0, CE // 16, scale, 0)
            pltpu.sync_copy(rows.at[p], acc.at[dstv.at[p]], add=True)

            @pl.when(i + 2 < NIT)
            def _():
                fetch_idx(i + 2, p)

        fetch_idx(0, 0)
        wait_idx(0)
        issue_gather(0)
        fetch_idx(1, 1)

        def body2(j, carry):
            chunk(j * 2, 0)
            chunk(j * 2 + 1, 1)
            return carry
        lax.fori_loop(0, NIT // 2, body2, 0)
        if NIT % 2 == 1:
            chunk(NIT - 1, 0)

        plsc.subcore_barrier()

        pltpu.sync_copy(denv, den_hbm.at[w])
        pltpu.sync_copy(acc.at[pl.ds(s * RP, RP)],
                        num_hbm.at[c, pl.ds(s * RP, RP)])

    return k(h, al, ar, src, dst)


def _gat_edges2(hflat, al2, ar2, src2, dst2, *, n, D, CE, ZR):
    E = src2.shape[1]
    EP = E // NS
    NIT = EP // CE
    RP = n // NS
    NZ = RP // ZR

    mesh = plsc.VectorSubcoreMesh(core_axis_name="c", subcore_axis_name="s",
                                  num_cores=NC, num_subcores=NS)

    @functools.partial(
        pl.kernel,
        out_type=[
            jax.ShapeDtypeStruct((NC, n, D), F32),
            jax.ShapeDtypeStruct((NC, NS, n), F32),
        ],
        mesh=mesh,
        compiler_params=pltpu.CompilerParams(
            needs_layout_passes=False, use_tc_tiling_on_sc=False),
        scratch_types=[
            pltpu.VMEM((n,), F32),
            pltpu.VMEM((n,), F32),
            pltpu.VMEM((n,), F32),
            pltpu.VMEM((2, CE), jnp.int32),
            pltpu.VMEM((2, CE), jnp.int32),
            pltpu.VMEM((2, CE), jnp.int32),
            pltpu.VMEM((CE,), F32),
            pltpu.VMEM((2, CE, D), F32),
            pltpu.VMEM_SHARED((n, D), F32),
            pltpu.SemaphoreType.DMA,
            pltpu.SemaphoreType.DMA,
            pltpu.SemaphoreType.DMA,
        ],
    )
    def k(h_hbm, al_hbm, ar_hbm, src_hbm, dst_hbm, num_hbm, den_hbm,
          alv, arv, denv, srcv, dstv, srca, exv, rows, acc, sem_i, sem_ga,
          sem_gb):
        c = lax.axis_index("c")
        s = lax.axis_index("s")
        zero16 = jnp.zeros((16,), F32)
        cnv = jnp.full((16,), c * n, jnp.int32)

        pltpu.sync_copy(al_hbm.at[c], alv)
        pltpu.sync_copy(ar_hbm.at[c], arv)

        def zzb(r, carry):
            for fg in range(D // 16):
                rows[0, r, pl.ds(fg * 16, 16)] = zero16
            return carry
        lax.fori_loop(0, ZR, zzb, 0)

        def zacc(i, carry):
            pltpu.async_copy(rows.at[0, pl.ds(0, ZR)],
                             acc.at[pl.ds(s * RP + i * ZR, ZR)], sem_ga)
            return carry
        lax.fori_loop(0, NZ, zacc, 0)

        def zden(i, carry):
            denv[pl.ds(i * 16, 16)] = zero16
            return carry
        lax.fori_loop(0, n // 16, zden, 0)

        neg = jnp.full((16,), -3e38, F32)

        def mx(i, carry):
            ca, cr = carry
            ca = jnp.maximum(ca, alv[pl.ds(i * 16, 16)])
            cr = jnp.maximum(cr, arv[pl.ds(i * 16, 16)])
            return ca, cr
        ca, cr = lax.fori_loop(0, n // 16, mx, (neg, neg))
        lanes = lax.iota(jnp.int32, 16)
        for kk in (8, 4, 2, 1):
            exv[pl.ds(0, 16)] = ca
            ca = jnp.maximum(ca, plsc.load_gather(exv, [lanes ^ kk]))
            exv[pl.ds(0, 16)] = cr
            cr = jnp.maximum(cr, plsc.load_gather(exv, [lanes ^ kk]))
        mv = jnp.maximum(ca + cr, jnp.zeros((16,), F32))

        def zdrain(i, carry):
            pltpu.make_async_copy(
                rows.at[0, pl.ds(0, ZR)],
                acc.at[pl.ds(s * RP + i * ZR, ZR)], sem_ga).wait()
            return carry
        lax.fori_loop(0, NZ, zdrain, 0)

        plsc.subcore_barrier()

        def fetch_idx(it, buf):
            base = s * EP + it * CE
            pltpu.async_copy(src_hbm.at[c, pl.ds(base, CE)],
                             srcv.at[buf], sem_i)
            pltpu.async_copy(dst_hbm.at[c, pl.ds(base, CE)],
                             dstv.at[buf], sem_i)

        def wait_idx(buf):
            pltpu.make_async_copy(
                src_hbm.at[0, pl.ds(0, CE)], srcv.at[buf], sem_i).wait()
            pltpu.make_async_copy(
                src_hbm.at[0, pl.ds(0, CE)], dstv.at[buf], sem_i).wait()
            for g in range(CE // 16):
                srca[buf, pl.ds(g * 16, 16)] = (
                    srcv[buf, pl.ds(g * 16, 16)] + cnv)

        gsems = (sem_ga, sem_gb)

        def issue_gather(p):
            pltpu.async_copy(h_hbm.at[srca.at[p]], rows.at[p], gsems[p])

        def wait_gather(p):
            pltpu.make_async_copy(
                h_hbm.at[srca.at[p]], rows.at[p], gsems[p]).wait()

        def chunk(i, p):
            q = 1 - p

            @pl.when(i + 1 < NIT)
            def _():
                wait_idx(q)
                issue_gather(q)
            for g in range(CE // 16):
                si = srcv[p, pl.ds(g * 16, 16)]
                di = dstv[p, pl.ds(g * 16, 16)]
                t = plsc.load_gather(alv, [si]) + plsc.load_gather(arv, [di])
                e = jnp.where(t > 0.0, t, 0.2 * t)
                ex = jnp.exp(e - mv)
                exv[pl.ds(g * 16, 16)] = ex
                plsc.addupdate_scatter(denv, [di], ex)
            wait_gather(p)

            def scale(g, carry):
                exg = exv[pl.ds(g * 16, 16)]
                for l in range(16):
                    eidx = g * 16 + l
                    exb = jnp.full((16,), exg[l], F32)
                    for fg in range(D // 16):
                        rows[p, eidx, pl.ds(fg * 16, 16)] = (
                            rows[p, eidx, pl.ds(fg * 16, 16)] * exb)
                return carry
            lax.fori_loop(0, CE // 16, scale, 0)
            pltpu.sync_copy(rows.at[p], acc.at[dstv.at[p]], add=True)

            @pl.when(i + 2 < NIT)
            def _():
                fetch_idx(i + 2, p)

        fetch_idx(0, 0)
        wait_idx(0)
        issue_gather(0)
        fetch_idx(1, 1)

        def body2(j, carry):
            chunk(j * 2, 0)
            chunk(j * 2 + 1, 1)
            return carry
        lax.fori_loop(0, NIT // 2, body2, 0)
        if NIT % 2 == 1:
            chunk(NIT - 1, 0)

        plsc.subcore_barrier()

        pltpu.sync_copy(denv, den_hbm.at[c, s])
        pltpu.sync_copy(acc.at[pl.ds(s * RP, RP)],
                        num_hbm.at[c, pl.ds(s * RP, RP)])

    return k(hflat, al2, ar2, src2, dst2)


def _gat_post(num, den, b, br):
    _, n, D = num.shape
    b2 = b[None, :]
    den_t = den.T

    def body(num_ref, den_ref, b_ref, out_ref):
        tot = num_ref[0] + num_ref[1]
        dd = jnp.sum(den_ref[...], axis=1)
        out_ref[...] = (tot / jnp.maximum(dd, 1e-30)[:, None]
                        + b_ref[0, :][None, :])

    return pl.pallas_call(
        body,
        grid=(n // br,),
        in_specs=[
            pl.BlockSpec((NC, br, D), lambda i: (0, i, 0)),
            pl.BlockSpec((br, NW), lambda i: (i, 0)),
            pl.BlockSpec((1, D), lambda i: (0, 0)),
        ],
        out_specs=pl.BlockSpec((br, D), lambda i: (i, 0)),
        out_shape=jax.ShapeDtypeStruct((n, D), F32),
    )(num, den_t, b2)


def _gat_post1(numg, dent, b, br):
    n, D = numg.shape

    def body(num_ref, den_ref, b_ref, out_ref):
        dd = jnp.sum(den_ref[...], axis=1)
        out_ref[...] = (num_ref[...] / jnp.maximum(dd, 1e-30)[:, None]
                        + b_ref[0, :][None, :])

    return pl.pallas_call(
        body,
        grid=(n // br,),
        in_specs=[
            pl.BlockSpec((br, D), lambda i: (i, 0)),
            pl.BlockSpec((br, NS), lambda i: (i, 0)),
            pl.BlockSpec((1, D), lambda i: (0, 0)),
        ],
        out_specs=pl.BlockSpec((br, D), lambda i: (i, 0)),
        out_shape=jax.ShapeDtypeStruct((n, D), F32),
    )(numg, dent, b[None, :])


def _matmul(a, b):
    n, k = a.shape
    d = b.shape[1]

    def body(a_ref, b_ref, o_ref):
        o_ref[...] = jnp.dot(a_ref[...], b_ref[...],
                             preferred_element_type=F32)

    return pl.pallas_call(
        body,
        out_shape=jax.ShapeDtypeStruct((n, d), F32),
    )(a, b)


def _biatt_a(F, Wk, Q, Sm, br):
    n, d = F.shape
    nt = Sm.shape[0]
    nb = n // br
    scale = 1.0 / math.sqrt(float(d))

    def body(f_ref, wk_ref, q_ref, sm_ref, out_ref, cm_ref):
        i = pl.program_id(0)
        K = jnp.dot(f_ref[...], wk_ref[...], preferred_element_type=F32)
        S = lax.dot_general(K, q_ref[...], (((1,), (1,)), ((), ())),
                            preferred_element_type=F32) * scale
        rm = jnp.max(S, axis=1)
        P = jnp.exp(S - rm[:, None])
        rs = jnp.sum(P, axis=1)
        out_ref[...] = f_ref[...] + (
            jnp.dot(P, sm_ref[...], preferred_element_type=F32)
            / rs[:, None])
        bm = jnp.max(S, axis=0)[None, :]

        @pl.when(i == 0)
        def _():
            cm_ref[...] = bm

        @pl.when(i > 0)
        def _():
            cm_ref[...] = jnp.maximum(cm_ref[...], bm)

    return pl.pallas_call(
        body,
        grid=(nb,),
        in_specs=[
            pl.BlockSpec((br, d), lambda i: (i, 0)),
            pl.BlockSpec((d, d), lambda i: (0, 0)),
            pl.BlockSpec((nt, d), lambda i: (0, 0)),
            pl.BlockSpec((nt, d), lambda i: (0, 0)),
        ],
        out_specs=[
            pl.BlockSpec((br, d), lambda i: (i, 0)),
            pl.BlockSpec((1, nt), lambda i: (0, 0)),
        ],
        out_shape=[
            jax.ShapeDtypeStruct((n, d), F32),
            jax.ShapeDtypeStruct((1, nt), F32),
        ],
    )(F, Wk, Q, Sm)


def _biatt_b(F, Wk, Q, Sm, colmax, br):
    n, d = F.shape
    nt = Sm.shape[0]
    nb = n // br
    scale = 1.0 / math.sqrt(float(d))

    def body(f_ref, wk_ref, q_ref, sm_ref, cm_ref, out_ref, nacc, cs):
        i = pl.program_id(0)

        @pl.when(i == 0)
        def _():
            nacc[...] = jnp.zeros_like(nacc)
            cs[...] = jnp.zeros_like(cs)

        K = jnp.dot(f_ref[...], wk_ref[...], preferred_element_type=F32)
        S = lax.dot_general(K, q_ref[...], (((1,), (1,)), ((), ())),
                            preferred_element_type=F32) * scale
        Eexp = jnp.exp(S - cm_ref[...])
        cs[...] += jnp.sum(Eexp, axis=0)[None, :]
        nacc[...] += lax.dot_general(Eexp, f_ref[...],
                                     (((0,), (0,)), ((), ())),
                                     preferred_element_type=F32)

        @pl.when(i == nb - 1)
        def _():
            out_ref[...] = sm_ref[...] + nacc[...] / cs[0, :][:, None]

    return pl.pallas_call(
        body,
        grid=(nb,),
        in_specs=[
            pl.BlockSpec((br, d), lambda i: (i, 0)),
            pl.BlockSpec((d, d), lambda i: (0, 0)),
            pl.BlockSpec((nt, d), lambda i: (0, 0)),
            pl.BlockSpec((nt, d), lambda i: (0, 0)),
            pl.BlockSpec((1, nt), lambda i: (0, 0)),
        ],
        out_specs=pl.BlockSpec((nt, d), lambda i: (0, 0)),
        out_shape=jax.ShapeDtypeStruct((nt, d), F32),
        scratch_shapes=[
            pltpu.VMEM((nt, d), F32),
            pltpu.VMEM((1, nt), F32),
        ],
    )(F, Wk, Q, Sm, colmax)


def _pool(x, batch8, br):
    n, D = x.shape
    nb = n // br

    def body(x_ref, b_ref, out_ref, sums, cnts):
        i = pl.program_id(0)

        @pl.when(i == 0)
        def _():
            sums[...] = jnp.zeros_like(sums)
            cnts[...] = jnp.zeros_like(cnts)

        ids = b_ref[:, 0]
        oh = (lax.broadcasted_iota(jnp.int32, (NB, br), 0)
              == ids[None, :]).astype(F32)
        sums[...] += jnp.dot(oh, x_ref[...], preferred_element_type=F32)
        cnts[...] += jnp.broadcast_to(jnp.sum(oh, axis=1)[:, None], (NB, D))

        @pl.when(i == nb - 1)
        def _():
            out_ref[...] = sums[...] / jnp.maximum(cnts[...], 1.0)

    return pl.pallas_call(
        body,
        grid=(nb,),
        in_specs=[
            pl.BlockSpec((br, D), lambda i: (i, 0)),
            pl.BlockSpec((br, 8), lambda i: (i, 0)),
        ],
        out_specs=pl.BlockSpec((NB, D), lambda i: (0, 0)),
        out_shape=jax.ShapeDtypeStruct((NB, D), F32),
        scratch_shapes=[
            pltpu.VMEM((NB, D), F32),
            pltpu.VMEM((NB, D), F32),
        ],
    )(x, batch8)


def _fusion(fused, W1, b1, W2p, b2p):
    in_dim, hid = W1.shape
    KB = 256
    nkb = hid // KB

    def body(f_ref, w1_ref, b1_ref, w2_ref, b2_ref, out_ref, acc):
        i = pl.program_id(0)

        @pl.when(i == 0)
        def _():
            acc[...] = jnp.broadcast_to(b2_ref[0, :][None, :], acc.shape)

        h1 = (jnp.dot(f_ref[...], w1_ref[...], preferred_element_type=F32)
              + b1_ref[0, :][None, :])
        acc[...] += jnp.dot(h1, w2_ref[...], preferred_element_type=F32)

        @pl.when(i == nkb - 1)
        def _():
            logits = acc[...]
            z = logits - jnp.max(logits, axis=1, keepdims=True)
            out_ref[...] = z - jnp.log(
                jnp.sum(jnp.exp(z), axis=1, keepdims=True))

    return pl.pallas_call(
        body,
        grid=(nkb,),
        in_specs=[
            pl.BlockSpec((NB, in_dim), lambda i: (0, 0)),
            pl.BlockSpec((in_dim, KB), lambda i: (0, i)),
            pl.BlockSpec((1, KB), lambda i: (0, i)),
            pl.BlockSpec((KB, NANS_PAD), lambda i: (i, 0)),
            pl.BlockSpec((1, NANS_PAD), lambda i: (0, 0)),
        ],
        out_specs=pl.BlockSpec((NB, NANS_PAD), lambda i: (0, 0)),
        out_shape=jax.ShapeDtypeStruct((NB, NANS_PAD), F32),
        scratch_shapes=[pltpu.VMEM((NB, NANS_PAD), F32)],
    )(fused, W1, b1, W2p, b2p)


def _gat_layer(x, src, dst, p, *, n, br, CE, ZR, pad_to=None):
    W, a_s, a_d, b = p
    if pad_to is not None and W.shape[1] < pad_to:
        extra = pad_to - W.shape[1]
        W = jnp.pad(W, ((0, 0), (0, extra)))
        a_s = jnp.pad(a_s, (0, extra))
        a_d = jnp.pad(a_d, (0, extra))
        b = jnp.pad(b, (0, extra))
    D = W.shape[1]
    h, aux = _dense_pre(x, W, a_s, a_d, br)
    num, den = _gat_edges(h, aux[:, 0], aux[:, 1], src, dst,
                          n=n, D=D, CE=CE, ZR=ZR)
    return _gat_post(num, den, b, br)


def kernel(params, word_syn_x, txt_syn_edge_index, txt_sem_x,
           txt_sem_edge_index, objects, obj_edge_index, obj_pos_x,
           txt_batch, obj_batch):
    syn, sem, objf, objp = word_syn_x, txt_sem_x, objects, obj_pos_x
    src2 = jnp.stack([txt_syn_edge_index[0], txt_sem_edge_index[0]])
    dst2 = jnp.stack([txt_syn_edge_index[1], txt_sem_edge_index[1]])
    ob_src, ob_dst = obj_edge_index[0], obj_edge_index[1]

    obj_kw = dict(n=OBJ_N, br=1000, CE=80, ZR=25)

    for lvl in range(3):
        Wsy, asy_s, asy_d, bsy = params['syn'][lvl]
        Wse, ase_s, ase_d, bse = params['sem'][lvl]
        h_sy, aux_sy = _dense_pre(syn, Wsy, asy_s, asy_d, 512)
        h_se, aux_se = _dense_pre(sem, Wse, ase_s, ase_d, 512)
        hflat = jnp.concatenate([h_sy, h_se], axis=0)
        al2 = jnp.stack([aux_sy[:, 0], aux_se[:, 0]])
        ar2 = jnp.stack([aux_sy[:, 1], aux_se[:, 1]])
        num2, den2 = _gat_edges2(hflat, al2, ar2, src2, dst2,
                                 n=TXT_N, D=128, CE=64, ZR=64)
        syn = _gat_post1(num2[0], den2[0].T, bsy, br=512)
        sem = _gat_post1(num2[1], den2[1].T, bse, br=512)
        objf = _gat_layer(objf, ob_src, ob_dst, params['objf'][lvl],
                          **obj_kw)
        objp = _gat_layer(objp, ob_src, ob_dst, params['objp'][lvl],
                          pad_to=16, **obj_kw)
        Wk, Wq = params['biatt'][lvl]
        Q = _matmul(sem, Wq)
        objf_new, colmax = _biatt_a(objf, Wk, Q, sem, br=1000)
        sem = _biatt_b(objf, Wk, Q, sem, colmax, br=1000)
        objf = objf_new

    txt_b8 = jnp.broadcast_to(txt_batch[:, None], (TXT_N, 8))
    obj_b8 = jnp.broadcast_to(obj_batch[:, None], (OBJ_N, 8))
    syn_p = _pool(syn, txt_b8, br=512)
    sem_p = _pool(sem, txt_b8, br=512)
    objf_p = _pool(objf, obj_b8, br=1000)
    objp_p = _pool(objp, obj_b8, br=1000)[:, :8]

    fused = jnp.concatenate([syn_p, objf_p, sem_p, objp_p], axis=1)
    W1, b1, W2, b2 = params['fusion']
    hid = W1.shape[1]
    hid_pad = ((hid + 255) // 256) * 256
    W1p = jnp.pad(W1, ((0, 0), (0, hid_pad - hid)))
    b1p = jnp.pad(b1, (0, hid_pad - hid))
    W2p = jnp.pad(W2, ((0, hid_pad - hid), (0, NANS_PAD - NANS)))
    b2p = jnp.pad(b2, (0, NANS_PAD - NANS), constant_values=-1e30)
    out = _fusion(fused, W1p, b1p[None, :], W2p, b2p[None, :])
    return out[:, :NANS]
